# Initial kernel scaffold; baseline (speedup 1.0000x reference)
import jax
import jax.numpy as jnp
from jax.experimental import pallas as pl


def kernel(x, edge_index, batch, W0, b0, g0, bt0, W1, b1, g1, bt1, W2, b2, g2, bt2, l1w, l1b, l2w, l2b):
    raise NotImplementedError("write your pallas kernel here")


# speedup vs baseline: 15.9019x; 15.9019x over previous
import functools

import jax
import jax.numpy as jnp
from jax import lax
from jax.experimental import pallas as pl
from jax.experimental.pallas import tpu as pltpu
from jax.experimental.pallas import tpu_sc as plsc

N = 10000
D = 128
E = 320000
G = 64
C = 10
EPS = 1e-5

NC, NS = 2, 16
NW = NC * NS
EPW = E // NW
K = 80
NCH = EPW // K
NP = 10240
RPT = NP // NS
NB = 1000
NBLK = N // NB
F32 = jnp.float32


def _sc_mesh():
    return plsc.VectorSubcoreMesh(
        core_axis_name="c", subcore_axis_name="s", num_cores=NC, num_subcores=NS)


def _deg_sc(dst3, ones16, z16):
    @functools.partial(
        pl.kernel,
        out_type=jax.ShapeDtypeStruct((NC, NP, 16), F32),
        mesh=_sc_mesh(),
        scratch_types=[
            pltpu.VMEM((NCH, K), jnp.int32),
            pltpu.VMEM((K, 16), F32),
            pltpu.VMEM_SHARED((NP, 16), F32),
        ],
    )
    def deg_kernel(dst_hbm, ones_hbm, z_hbm, out_hbm, idx_v, ones_v, acc_sh):
        c = lax.axis_index("c")
        s = lax.axis_index("s")
        w = c * NS + s
        pltpu.sync_copy(z_hbm, acc_sh.at[pl.ds(s * RPT, RPT)])
        pltpu.sync_copy(ones_hbm, ones_v)
        pltpu.sync_copy(dst_hbm.at[w], idx_v)
        plsc.subcore_barrier()

        def body(j, carry):
            pltpu.sync_copy(ones_v, acc_sh.at[idx_v.at[j]], add=True)
            return carry

        lax.fori_loop(0, NCH, body, 0)
        plsc.subcore_barrier()
        pltpu.sync_copy(acc_sh.at[pl.ds(s * RPT, RPT)],
                        out_hbm.at[c].at[pl.ds(s * RPT, RPT)])

    return deg_kernel(dst3, ones16, z16)


def _scatter_sc(h, src3, dst3, z128):
    @functools.partial(
        pl.kernel,
        out_type=jax.ShapeDtypeStruct((NC, NP, D), F32),
        mesh=_sc_mesh(),
        scratch_types=[
            pltpu.VMEM((NCH, K), jnp.int32),
            pltpu.VMEM((NCH, K), jnp.int32),
            pltpu.VMEM((K, D), F32),
            pltpu.VMEM_SHARED((NP, D), F32),
            pltpu.SemaphoreType.DMA,
        ],
    )
    def scat_kernel(h_hbm, src_hbm, dst_hbm, z_hbm, out_hbm,
                    srcv, dstv, rows, acc_sh, sem):
        c = lax.axis_index("c")
        s = lax.axis_index("s")
        w = c * NS + s
        pltpu.sync_copy(z_hbm, acc_sh.at[pl.ds(s * RPT, RPT)])
        pltpu.sync_copy(src_hbm.at[w], srcv)
        pltpu.sync_copy(dst_hbm.at[w], dstv)
        plsc.subcore_barrier()

        def body(j, carry):
            pltpu.async_copy(h_hbm.at[srcv.at[j]], rows, sem).wait()
            pltpu.sync_copy(rows, acc_sh.at[dstv.at[j]], add=True)
            return carry

        lax.fori_loop(0, NCH, body, 0)
        plsc.subcore_barrier()
        pltpu.sync_copy(acc_sh.at[pl.ds(s * RPT, RPT)],
                        out_hbm.at[c].at[pl.ds(s * RPT, RPT)])

    return scat_kernel(h, src3, dst3, z128)


def _mm0(x, W, cnt):
    def body(x_ref, w_ref, cnt_ref, hp_ref, dinv_ref):
        deg = cnt_ref[0, :, 0:1] + cnt_ref[1, :, 0:1] + 1.0
        dv = lax.rsqrt(deg)
        h = jnp.dot(x_ref[...], w_ref[...], preferred_element_type=F32)
        hp_ref[...] = h * dv
        dinv_ref[...] = jnp.broadcast_to(dv, (NB, 16))

    return pl.pallas_call(
        body,
        grid=(NBLK,),
        in_specs=[
            pl.BlockSpec((NB, D), lambda i: (i, 0)),
            pl.BlockSpec((D, D), lambda i: (0, 0)),
            pl.BlockSpec((NC, NB, 16), lambda i: (0, i, 0)),
        ],
        out_specs=[
            pl.BlockSpec((NB, D), lambda i: (i, 0)),
            pl.BlockSpec((NB, 16), lambda i: (i, 0)),
        ],
        out_shape=[
            jax.ShapeDtypeStruct((N, D), F32),
            jax.ShapeDtypeStruct((N, 16), F32),
        ],
    )(x, W, cnt)


def _combine(acc, hp, dinv, b):
    def body(acc_ref, hp_ref, dinv_ref, b_ref, t_ref, st_ref):
        i = pl.program_id(0)
        dv = dinv_ref[:, 0:1]
        o = dv * (acc_ref[0] + acc_ref[1] + hp_ref[...]) + b_ref[...]
        t = jnp.where(o > 0, o, jnp.exp(o) - 1.0)
        t_ref[...] = t

        @pl.when(i == 0)
        def _():
            st_ref[...] = jnp.zeros_like(st_ref)

        st_ref[0:1, :] += jnp.sum(t, axis=0, keepdims=True)
        st_ref[1:2, :] += jnp.sum(t * t, axis=0, keepdims=True)

    return pl.pallas_call(
        body,
        grid=(NBLK,),
        in_specs=[
            pl.BlockSpec((NC, NB, D), lambda i: (0, i, 0)),
            pl.BlockSpec((NB, D), lambda i: (i, 0)),
            pl.BlockSpec((NB, 16), lambda i: (i, 0)),
            pl.BlockSpec((1, D), lambda i: (0, 0)),
        ],
        out_specs=[
            pl.BlockSpec((NB, D), lambda i: (i, 0)),
            pl.BlockSpec((8, D), lambda i: (0, 0)),
        ],
        out_shape=[
            jax.ShapeDtypeStruct((N, D), F32),
            jax.ShapeDtypeStruct((8, D), F32),
        ],
    )(acc, hp, dinv, b)


def _mm_bn(t, st, g, bt, W, dinv):
    def body(t_ref, st_ref, g_ref, bt_ref, w_ref, dinv_ref, hp_ref):
        m = st_ref[0:1, :] / N
        v = st_ref[1:2, :] / N - m * m
        sc = g_ref[...] * lax.rsqrt(v + EPS)
        sh = bt_ref[...] - m * sc
        y = t_ref[...] * sc + sh
        h = jnp.dot(y, w_ref[...], preferred_element_type=F32)
        hp_ref[...] = h * dinv_ref[:, 0:1]

    return pl.pallas_call(
        body,
        grid=(NBLK,),
        in_specs=[
            pl.BlockSpec((NB, D), lambda i: (i, 0)),
            pl.BlockSpec((8, D), lambda i: (0, 0)),
            pl.BlockSpec((1, D), lambda i: (0, 0)),
            pl.BlockSpec((1, D), lambda i: (0, 0)),
            pl.BlockSpec((D, D), lambda i: (0, 0)),
            pl.BlockSpec((NB, 16), lambda i: (i, 0)),
        ],
        out_specs=pl.BlockSpec((NB, D), lambda i: (i, 0)),
        out_shape=jax.ShapeDtypeStruct((N, D), F32),
    )(t, st, g, bt, W, dinv)


def _pool_head(t, st, g, bt, batch2, l1w, l1b, l2w, l2b):
    def body(t_ref, st_ref, g_ref, bt_ref, b_ref, l1w_ref, l1b_ref,
             l2w_ref, l2b_ref, out_ref, pool_sc, cnt_sc):
        i = pl.program_id(0)
        m = st_ref[0:1, :] / N
        v = st_ref[1:2, :] / N - m * m
        sc = g_ref[...] * lax.rsqrt(v + EPS)
        sh = bt_ref[...] - m * sc
        y = t_ref[...] * sc + sh
        gids = lax.broadcasted_iota(jnp.int32, (1, G), 1)
        oh = (b_ref[...] == gids).astype(F32)

        @pl.when(i == 0)
        def _():
            pool_sc[...] = jnp.zeros_like(pool_sc)
            cnt_sc[...] = jnp.zeros_like(cnt_sc)

        pool_sc[...] += lax.dot_general(
            oh, y, (((0,), (0,)), ((), ())), preferred_element_type=F32)
        cnt_sc[...] += lax.dot_general(
            oh, jnp.ones((NB, 1), F32), (((0,), (0,)), ((), ())),
            preferred_element_type=F32)

        @pl.when(i == pl.num_programs(0) - 1)
        def _():
            cnt = jnp.maximum(cnt_sc[...], 1.0)
            pooled = pool_sc[...] / cnt
            z = jnp.dot(pooled, l1w_ref[...], preferred_element_type=F32)
            z = jnp.maximum(z + l1b_ref[...], 0.0)
            z2 = jnp.dot(z, l2w_ref[...], preferred_element_type=F32)
            z2 = z2 + l2b_ref[...]
            mx = jnp.max(z2, axis=-1, keepdims=True)
            lse = jnp.log(jnp.sum(jnp.exp(z2 - mx), axis=-1, keepdims=True)) + mx
            out_ref[...] = z2 - lse

    return pl.pallas_call(
        body,
        grid=(NBLK,),
        in_specs=[
            pl.BlockSpec((NB, D), lambda i: (i, 0)),
            pl.BlockSpec((8, D), lambda i: (0, 0)),
            pl.BlockSpec((1, D), lambda i: (0, 0)),
            pl.BlockSpec((1, D), lambda i: (0, 0)),
            pl.BlockSpec((NB, 1), lambda i: (i, 0)),
            pl.BlockSpec((D, D), lambda i: (0, 0)),
            pl.BlockSpec((1, D), lambda i: (0, 0)),
            pl.BlockSpec((D, C), lambda i: (0, 0)),
            pl.BlockSpec((1, C), lambda i: (0, 0)),
        ],
        out_specs=pl.BlockSpec((G, C), lambda i: (0, 0)),
        out_shape=jax.ShapeDtypeStruct((G, C), F32),
        scratch_shapes=[
            pltpu.VMEM((G, D), F32),
            pltpu.VMEM((G, 1), F32),
        ],
    )(t, st, g, bt, batch2, l1w, l1b, l2w, l2b)


def kernel(x, edge_index, batch, W0, b0, g0, bt0, W1, b1, g1, bt1,
           W2, b2, g2, bt2, l1w, l1b, l2w, l2b):
    src3 = edge_index[0].reshape(NW, NCH, K)
    dst3 = edge_index[1].reshape(NW, NCH, K)
    ones16 = jnp.concatenate(
        [jnp.ones((K, 1), F32), jnp.zeros((K, 15), F32)], axis=1)
    z16 = jnp.zeros((RPT, 16), F32)
    z128 = jnp.zeros((RPT, D), F32)
    batch2 = batch.reshape(N, 1)

    cnt = _deg_sc(dst3, ones16, z16)
    hp, dinv = _mm0(x, W0, cnt)

    layer = [(b0, g0, bt0), (b1, g1, bt1), (b2, g2, bt2)]
    nextW = [W1, W2]
    t = st = None
    for li, (b, g, bt) in enumerate(layer):
        acc = _scatter_sc(hp, src3, dst3, z128)
        t, st = _combine(acc, hp, dinv, b.reshape(1, D))
        if li < 2:
            hp = _mm_bn(t, st, g.reshape(1, D), bt.reshape(1, D),
                        nextW[li], dinv)

    return _pool_head(t, st, g2.reshape(1, D), bt2.reshape(1, D),
                      batch2, l1w, l1b.reshape(1, D), l2w, l2b.reshape(1, C))


# speedup vs baseline: 22.8353x; 1.4360x over previous
import functools

import jax
import jax.numpy as jnp
from jax import lax
from jax.experimental import pallas as pl
from jax.experimental.pallas import tpu as pltpu
from jax.experimental.pallas import tpu_sc as plsc

N = 10000
D = 128
E = 320000
G = 64
C = 10
EPS = 1e-5

NC, NS = 2, 16
NW = NC * NS
EPW = E // NW
K = 80
NCH = EPW // K
NGRP, CPG = 5, 25
NP = 10240
RPT = NP // NS
NB = 1000
NBLK = N // NB
F32 = jnp.float32


def _sc_mesh():
    return plsc.VectorSubcoreMesh(
        core_axis_name="c", subcore_axis_name="s", num_cores=NC, num_subcores=NS)


def _deg_sc(dst3, ones16, z16):
    @functools.partial(
        pl.kernel,
        out_type=jax.ShapeDtypeStruct((NC, NP, 16), F32),
        mesh=_sc_mesh(),
        scratch_types=[
            pltpu.VMEM((CPG, K), jnp.int32),
            pltpu.VMEM((K, 16), F32),
            pltpu.VMEM_SHARED((NP, 16), F32),
        ],
    )
    def deg_kernel(dst_hbm, ones_hbm, z_hbm, out_hbm, idx_v, ones_v, acc_sh):
        c = lax.axis_index("c")
        s = lax.axis_index("s")
        w = c * NS + s
        pltpu.sync_copy(z_hbm, acc_sh.at[pl.ds(s * RPT, RPT)])
        pltpu.sync_copy(ones_hbm, ones_v)
        plsc.subcore_barrier()

        def group(gi, carry):
            pltpu.sync_copy(dst_hbm.at[w, gi], idx_v)

            def body(j, c2):
                pltpu.sync_copy(ones_v, acc_sh.at[idx_v.at[j]], add=True)
                return c2

            lax.fori_loop(0, CPG, body, 0)
            return carry

        lax.fori_loop(0, NGRP, group, 0)
        plsc.subcore_barrier()
        pltpu.sync_copy(acc_sh.at[pl.ds(s * RPT, RPT)],
                        out_hbm.at[c].at[pl.ds(s * RPT, RPT)])

    return deg_kernel(dst3, ones16, z16)


def _scatter_sc(h, src3, dst3, z128):
    @functools.partial(
        pl.kernel,
        out_type=jax.ShapeDtypeStruct((NC, NP, D), F32),
        mesh=_sc_mesh(),
        scratch_types=[
            pltpu.VMEM((CPG, K), jnp.int32),
            pltpu.VMEM((CPG, K), jnp.int32),
            pltpu.VMEM((K, D), F32),
            pltpu.VMEM((K, D), F32),
            pltpu.VMEM_SHARED((NP, D), F32),
            pltpu.SemaphoreType.DMA,
            pltpu.SemaphoreType.DMA,
        ],
    )
    def scat_kernel(h_hbm, src_hbm, dst_hbm, z_hbm, out_hbm,
                    srcv, dstv, rows_a, rows_b, acc_sh, sem_a, sem_b):
        c = lax.axis_index("c")
        s = lax.axis_index("s")
        w = c * NS + s
        pltpu.sync_copy(z_hbm, acc_sh.at[pl.ds(s * RPT, RPT)])
        plsc.subcore_barrier()

        def g_start(j, buf, sem):
            pltpu.async_copy(h_hbm.at[srcv.at[j]], buf, sem)

        def g_wait(buf, sem):
            pltpu.make_async_copy(h_hbm.at[srcv.at[0]], buf, sem).wait()

        def scat(j, buf):
            pltpu.sync_copy(buf, acc_sh.at[dstv.at[j]], add=True)

        def group(gi, carry):
            pltpu.sync_copy(src_hbm.at[w, gi], srcv)
            pltpu.sync_copy(dst_hbm.at[w, gi], dstv)
            g_start(0, rows_a, sem_a)

            def body(k, c2):
                j = 2 * k
                g_start(j + 1, rows_b, sem_b)
                g_wait(rows_a, sem_a)
                scat(j, rows_a)
                g_start(j + 2, rows_a, sem_a)
                g_wait(rows_b, sem_b)
                scat(j + 1, rows_b)
                return c2

            lax.fori_loop(0, (CPG - 1) // 2, body, 0)
            g_wait(rows_a, sem_a)
            scat(CPG - 1, rows_a)
            return carry

        lax.fori_loop(0, NGRP, group, 0)
        plsc.subcore_barrier()
        pltpu.sync_copy(acc_sh.at[pl.ds(s * RPT, RPT)],
                        out_hbm.at[c].at[pl.ds(s * RPT, RPT)])

    return scat_kernel(h, src3, dst3, z128)


def _mm0(x, W, cnt):
    def body(x_ref, w_ref, cnt_ref, hp_ref, dinv_ref):
        deg = cnt_ref[0, :, 0:1] + cnt_ref[1, :, 0:1] + 1.0
        dv = lax.rsqrt(deg)
        h = jnp.dot(x_ref[...], w_ref[...], preferred_element_type=F32)
        hp_ref[...] = h * dv
        dinv_ref[...] = jnp.broadcast_to(dv, (NB, 16))

    return pl.pallas_call(
        body,
        grid=(NBLK,),
        in_specs=[
            pl.BlockSpec((NB, D), lambda i: (i, 0)),
            pl.BlockSpec((D, D), lambda i: (0, 0)),
            pl.BlockSpec((NC, NB, 16), lambda i: (0, i, 0)),
        ],
        out_specs=[
            pl.BlockSpec((NB, D), lambda i: (i, 0)),
            pl.BlockSpec((NB, 16), lambda i: (i, 0)),
        ],
        out_shape=[
            jax.ShapeDtypeStruct((N, D), F32),
            jax.ShapeDtypeStruct((N, 16), F32),
        ],
    )(x, W, cnt)


def _combine(acc, hp, dinv, b):
    def body(acc_ref, hp_ref, dinv_ref, b_ref, t_ref, st_ref):
        i = pl.program_id(0)
        dv = dinv_ref[:, 0:1]
        o = dv * (acc_ref[0] + acc_ref[1] + hp_ref[...]) + b_ref[...]
        t = jnp.where(o > 0, o, jnp.exp(o) - 1.0)
        t_ref[...] = t

        @pl.when(i == 0)
        def _():
            st_ref[...] = jnp.zeros_like(st_ref)

        st_ref[0:1, :] += jnp.sum(t, axis=0, keepdims=True)
        st_ref[1:2, :] += jnp.sum(t * t, axis=0, keepdims=True)

    return pl.pallas_call(
        body,
        grid=(NBLK,),
        in_specs=[
            pl.BlockSpec((NC, NB, D), lambda i: (0, i, 0)),
            pl.BlockSpec((NB, D), lambda i: (i, 0)),
            pl.BlockSpec((NB, 16), lambda i: (i, 0)),
            pl.BlockSpec((1, D), lambda i: (0, 0)),
        ],
        out_specs=[
            pl.BlockSpec((NB, D), lambda i: (i, 0)),
            pl.BlockSpec((8, D), lambda i: (0, 0)),
        ],
        out_shape=[
            jax.ShapeDtypeStruct((N, D), F32),
            jax.ShapeDtypeStruct((8, D), F32),
        ],
    )(acc, hp, dinv, b)


def _mm_bn(t, st, g, bt, W, dinv):
    def body(t_ref, st_ref, g_ref, bt_ref, w_ref, dinv_ref, hp_ref):
        m = st_ref[0:1, :] / N
        v = st_ref[1:2, :] / N - m * m
        sc = g_ref[...] * lax.rsqrt(v + EPS)
        sh = bt_ref[...] - m * sc
        y = t_ref[...] * sc + sh
        h = jnp.dot(y, w_ref[...], preferred_element_type=F32)
        hp_ref[...] = h * dinv_ref[:, 0:1]

    return pl.pallas_call(
        body,
        grid=(NBLK,),
        in_specs=[
            pl.BlockSpec((NB, D), lambda i: (i, 0)),
            pl.BlockSpec((8, D), lambda i: (0, 0)),
            pl.BlockSpec((1, D), lambda i: (0, 0)),
            pl.BlockSpec((1, D), lambda i: (0, 0)),
            pl.BlockSpec((D, D), lambda i: (0, 0)),
            pl.BlockSpec((NB, 16), lambda i: (i, 0)),
        ],
        out_specs=pl.BlockSpec((NB, D), lambda i: (i, 0)),
        out_shape=jax.ShapeDtypeStruct((N, D), F32),
    )(t, st, g, bt, W, dinv)


def _pool_head(t, st, g, bt, batch2, l1w, l1b, l2w, l2b):
    def body(t_ref, st_ref, g_ref, bt_ref, b_ref, l1w_ref, l1b_ref,
             l2w_ref, l2b_ref, out_ref, pool_sc, cnt_sc):
        i = pl.program_id(0)
        m = st_ref[0:1, :] / N
        v = st_ref[1:2, :] / N - m * m
        sc = g_ref[...] * lax.rsqrt(v + EPS)
        sh = bt_ref[...] - m * sc
        y = t_ref[...] * sc + sh
        gids = lax.broadcasted_iota(jnp.int32, (1, G), 1)
        oh = (b_ref[...] == gids).astype(F32)

        @pl.when(i == 0)
        def _():
            pool_sc[...] = jnp.zeros_like(pool_sc)
            cnt_sc[...] = jnp.zeros_like(cnt_sc)

        pool_sc[...] += lax.dot_general(
            oh, y, (((0,), (0,)), ((), ())), preferred_element_type=F32)
        cnt_sc[...] += lax.dot_general(
            oh, jnp.ones((NB, 1), F32), (((0,), (0,)), ((), ())),
            preferred_element_type=F32)

        @pl.when(i == pl.num_programs(0) - 1)
        def _():
            cnt = jnp.maximum(cnt_sc[...], 1.0)
            pooled = pool_sc[...] / cnt
            z = jnp.dot(pooled, l1w_ref[...], preferred_element_type=F32)
            z = jnp.maximum(z + l1b_ref[...], 0.0)
            z2 = jnp.dot(z, l2w_ref[...], preferred_element_type=F32)
            z2 = z2 + l2b_ref[...]
            mx = jnp.max(z2, axis=-1, keepdims=True)
            lse = jnp.log(jnp.sum(jnp.exp(z2 - mx), axis=-1, keepdims=True)) + mx
            out_ref[...] = z2 - lse

    return pl.pallas_call(
        body,
        grid=(NBLK,),
        in_specs=[
            pl.BlockSpec((NB, D), lambda i: (i, 0)),
            pl.BlockSpec((8, D), lambda i: (0, 0)),
            pl.BlockSpec((1, D), lambda i: (0, 0)),
            pl.BlockSpec((1, D), lambda i: (0, 0)),
            pl.BlockSpec((NB, 1), lambda i: (i, 0)),
            pl.BlockSpec((D, D), lambda i: (0, 0)),
            pl.BlockSpec((1, D), lambda i: (0, 0)),
            pl.BlockSpec((D, C), lambda i: (0, 0)),
            pl.BlockSpec((1, C), lambda i: (0, 0)),
        ],
        out_specs=pl.BlockSpec((G, C), lambda i: (0, 0)),
        out_shape=jax.ShapeDtypeStruct((G, C), F32),
        scratch_shapes=[
            pltpu.VMEM((G, D), F32),
            pltpu.VMEM((G, 1), F32),
        ],
    )(t, st, g, bt, batch2, l1w, l1b, l2w, l2b)


def kernel(x, edge_index, batch, W0, b0, g0, bt0, W1, b1, g1, bt1,
           W2, b2, g2, bt2, l1w, l1b, l2w, l2b):
    src3 = edge_index[0].reshape(NW, NGRP, CPG, K)
    dst3 = edge_index[1].reshape(NW, NGRP, CPG, K)
    ones16 = jnp.concatenate(
        [jnp.ones((K, 1), F32), jnp.zeros((K, 15), F32)], axis=1)
    z16 = jnp.zeros((RPT, 16), F32)
    z128 = jnp.zeros((RPT, D), F32)
    batch2 = batch.reshape(N, 1)

    cnt = _deg_sc(dst3, ones16, z16)
    hp, dinv = _mm0(x, W0, cnt)

    layer = [(b0, g0, bt0), (b1, g1, bt1), (b2, g2, bt2)]
    nextW = [W1, W2]
    t = st = None
    for li, (b, g, bt) in enumerate(layer):
        acc = _scatter_sc(hp, src3, dst3, z128)
        t, st = _combine(acc, hp, dinv, b.reshape(1, D))
        if li < 2:
            hp = _mm_bn(t, st, g.reshape(1, D), bt.reshape(1, D),
                        nextW[li], dinv)

    return _pool_head(t, st, g2.reshape(1, D), bt2.reshape(1, D),
                      batch2, l1w, l1b.reshape(1, D), l2w, l2b.reshape(1, C))


# speedup vs baseline: 22.9982x; 1.0071x over previous
import functools

import jax
import jax.numpy as jnp
from jax import lax
from jax.experimental import pallas as pl
from jax.experimental.pallas import tpu as pltpu
from jax.experimental.pallas import tpu_sc as plsc

N = 10000
D = 128
E = 320000
G = 64
C = 10
EPS = 1e-5

NC, NS = 2, 16
NW = NC * NS
EPW = E // NW
K = 80
NCH = EPW // K
NGRP, CPG = 5, 25
NP = 10240
RPT = NP // NS
NB = 1000
NBLK = N // NB
F32 = jnp.float32


def _sc_mesh():
    return plsc.VectorSubcoreMesh(
        core_axis_name="c", subcore_axis_name="s", num_cores=NC, num_subcores=NS)


def _deg_sc(dst3, ones16, z16):
    @functools.partial(
        pl.kernel,
        out_type=jax.ShapeDtypeStruct((NC, NP, 16), F32),
        mesh=_sc_mesh(),
        scratch_types=[
            pltpu.VMEM((CPG, K), jnp.int32),
            pltpu.VMEM((K, 16), F32),
            pltpu.VMEM_SHARED((NP, 16), F32),
        ],
    )
    def deg_kernel(dst_hbm, ones_hbm, z_hbm, out_hbm, idx_v, ones_v, acc_sh):
        c = lax.axis_index("c")
        s = lax.axis_index("s")
        w = c * NS + s
        pltpu.sync_copy(z_hbm, acc_sh.at[pl.ds(s * RPT, RPT)])
        pltpu.sync_copy(ones_hbm, ones_v)
        plsc.subcore_barrier()

        def group(gi, carry):
            pltpu.sync_copy(dst_hbm.at[w, gi], idx_v)

            def body(j, c2):
                pltpu.sync_copy(ones_v, acc_sh.at[idx_v.at[j]], add=True)
                return c2

            lax.fori_loop(0, CPG, body, 0)
            return carry

        lax.fori_loop(0, NGRP, group, 0)
        plsc.subcore_barrier()
        pltpu.sync_copy(acc_sh.at[pl.ds(s * RPT, RPT)],
                        out_hbm.at[c].at[pl.ds(s * RPT, RPT)])

    return deg_kernel(dst3, ones16, z16)


def _scatter_sc(h, src3, dst3, z128):
    @functools.partial(
        pl.kernel,
        out_type=jax.ShapeDtypeStruct((NC, NP, D), F32),
        mesh=_sc_mesh(),
        scratch_types=[
            pltpu.VMEM((CPG, K), jnp.int32),
            pltpu.VMEM((CPG, K), jnp.int32),
            pltpu.VMEM((K, D), F32),
            pltpu.VMEM((K, D), F32),
            pltpu.VMEM_SHARED((NP, D), F32),
            pltpu.SemaphoreType.DMA,
            pltpu.SemaphoreType.DMA,
        ],
    )
    def scat_kernel(h_hbm, src_hbm, dst_hbm, z_hbm, out_hbm,
                    srcv, dstv, rows_a, rows_b, acc_sh, sem_a, sem_b):
        c = lax.axis_index("c")
        s = lax.axis_index("s")
        w = c * NS + s
        pltpu.sync_copy(z_hbm, acc_sh.at[pl.ds(s * RPT, RPT)])
        plsc.subcore_barrier()

        def g_start(j, buf, sem):
            pltpu.async_copy(h_hbm.at[srcv.at[j]], buf, sem)

        def g_wait(buf, sem):
            pltpu.make_async_copy(h_hbm.at[srcv.at[0]], buf, sem).wait()

        def scat(j, buf):
            pltpu.sync_copy(buf, acc_sh.at[dstv.at[j]], add=True)

        def group(gi, carry):
            pltpu.sync_copy(src_hbm.at[w, gi], srcv)
            pltpu.sync_copy(dst_hbm.at[w, gi], dstv)
            g_start(0, rows_a, sem_a)

            def body(k, c2):
                j = 2 * k
                g_start(j + 1, rows_b, sem_b)
                g_wait(rows_a, sem_a)
                scat(j, rows_a)
                g_start(j + 2, rows_a, sem_a)
                g_wait(rows_b, sem_b)
                scat(j + 1, rows_b)
                return c2

            lax.fori_loop(0, (CPG - 1) // 2, body, 0)
            g_wait(rows_a, sem_a)
            scat(CPG - 1, rows_a)
            return carry

        lax.fori_loop(0, NGRP, group, 0)
        plsc.subcore_barrier()
        pltpu.sync_copy(acc_sh.at[pl.ds(s * RPT, RPT)],
                        out_hbm.at[c].at[pl.ds(s * RPT, RPT)])

    return scat_kernel(h, src3, dst3, z128)


def _mm0(x, W, cnt):
    def body(x_ref, w_ref, cnt_ref, hp_ref, dinv_ref):
        deg = cnt_ref[0, :, 0:1] + cnt_ref[1, :, 0:1] + 1.0
        dv = lax.rsqrt(deg)
        h = jnp.dot(x_ref[...], w_ref[...], preferred_element_type=F32)
        hp_ref[...] = h * dv
        dinv_ref[...] = jnp.broadcast_to(dv, (NB, 16))

    return pl.pallas_call(
        body,
        grid=(NBLK,),
        in_specs=[
            pl.BlockSpec((NB, D), lambda i: (i, 0)),
            pl.BlockSpec((D, D), lambda i: (0, 0)),
            pl.BlockSpec((NC, NB, 16), lambda i: (0, i, 0)),
        ],
        out_specs=[
            pl.BlockSpec((NB, D), lambda i: (i, 0)),
            pl.BlockSpec((NB, 16), lambda i: (i, 0)),
        ],
        out_shape=[
            jax.ShapeDtypeStruct((N, D), F32),
            jax.ShapeDtypeStruct((N, 16), F32),
        ],
    )(x, W, cnt)


def _combine_mm(acc, hp, dinv, b, g, bt, W):
    def body(acc_ref, hp_ref, dinv_ref, b_ref, g_ref, bt_ref, w_ref,
             out_ref, t_sc, st_sc):
        p = pl.program_id(0)
        i = pl.program_id(1)

        @pl.when((p == 0) & (i == 0))
        def _():
            st_sc[...] = jnp.zeros_like(st_sc)

        @pl.when(p == 0)
        def _():
            dv = dinv_ref[:, 0:1]
            o = dv * (acc_ref[0] + acc_ref[1] + hp_ref[...]) + b_ref[...]
            t = jnp.where(o > 0, o, jnp.exp(o) - 1.0)
            t_sc[i] = t
            st_sc[0:1, :] += jnp.sum(t, axis=0, keepdims=True)
            st_sc[1:2, :] += jnp.sum(t * t, axis=0, keepdims=True)

        @pl.when(p == 1)
        def _():
            m = st_sc[0:1, :] / N
            v = st_sc[1:2, :] / N - m * m
            sc = g_ref[...] * lax.rsqrt(v + EPS)
            sh = bt_ref[...] - m * sc
            y = t_sc[i] * sc + sh
            h = jnp.dot(y, w_ref[...], preferred_element_type=F32)
            out_ref[...] = h * dinv_ref[:, 0:1]

    return pl.pallas_call(
        body,
        grid=(2, NBLK),
        in_specs=[
            pl.BlockSpec((NC, NB, D), lambda p, i: (0, i * (1 - p), 0)),
            pl.BlockSpec((NB, D), lambda p, i: (i * (1 - p), 0)),
            pl.BlockSpec((NB, 16), lambda p, i: (i, 0)),
            pl.BlockSpec((1, D), lambda p, i: (0, 0)),
            pl.BlockSpec((1, D), lambda p, i: (0, 0)),
            pl.BlockSpec((1, D), lambda p, i: (0, 0)),
            pl.BlockSpec((D, D), lambda p, i: (0, 0)),
        ],
        out_specs=pl.BlockSpec((NB, D), lambda p, i: (i, 0)),
        out_shape=jax.ShapeDtypeStruct((N, D), F32),
        scratch_shapes=[
            pltpu.VMEM((NBLK, NB, D), F32),
            pltpu.VMEM((8, D), F32),
        ],
    )(acc, hp, dinv, b, g, bt, W)


def _combine_pool(acc, hp, dinv, b, g, bt, batch2, l1w, l1b, l2w, l2b):
    def body(acc_ref, hp_ref, dinv_ref, b_ref, g_ref, bt_ref, batch_ref,
             l1w_ref, l1b_ref, l2w_ref, l2b_ref, out_ref,
             t_sc, st_sc, pool_sc, cnt_sc):
        p = pl.program_id(0)
        i = pl.program_id(1)

        @pl.when((p == 0) & (i == 0))
        def _():
            st_sc[...] = jnp.zeros_like(st_sc)
            pool_sc[...] = jnp.zeros_like(pool_sc)
            cnt_sc[...] = jnp.zeros_like(cnt_sc)

        @pl.when(p == 0)
        def _():
            dv = dinv_ref[:, 0:1]
            o = dv * (acc_ref[0] + acc_ref[1] + hp_ref[...]) + b_ref[...]
            t = jnp.where(o > 0, o, jnp.exp(o) - 1.0)
            t_sc[i] = t
            st_sc[0:1, :] += jnp.sum(t, axis=0, keepdims=True)
            st_sc[1:2, :] += jnp.sum(t * t, axis=0, keepdims=True)

        @pl.when(p == 1)
        def _():
            m = st_sc[0:1, :] / N
            v = st_sc[1:2, :] / N - m * m
            sc = g_ref[...] * lax.rsqrt(v + EPS)
            sh = bt_ref[...] - m * sc
            y = t_sc[i] * sc + sh
            gids = lax.broadcasted_iota(jnp.int32, (1, G), 1)
            oh = (batch_ref[...] == gids).astype(F32)
            pool_sc[...] += lax.dot_general(
                oh, y, (((0,), (0,)), ((), ())), preferred_element_type=F32)
            cnt_sc[...] += lax.dot_general(
                oh, jnp.ones((NB, 1), F32), (((0,), (0,)), ((), ())),
                preferred_element_type=F32)

        @pl.when((p == 1) & (i == NBLK - 1))
        def _():
            cnt = jnp.maximum(cnt_sc[...], 1.0)
            pooled = pool_sc[...] / cnt
            z = jnp.dot(pooled, l1w_ref[...], preferred_element_type=F32)
            z = jnp.maximum(z + l1b_ref[...], 0.0)
            z2 = jnp.dot(z, l2w_ref[...], preferred_element_type=F32)
            z2 = z2 + l2b_ref[...]
            mx = jnp.max(z2, axis=-1, keepdims=True)
            lse = jnp.log(jnp.sum(jnp.exp(z2 - mx), axis=-1, keepdims=True)) + mx
            out_ref[...] = z2 - lse

    return pl.pallas_call(
        body,
        grid=(2, NBLK),
        in_specs=[
            pl.BlockSpec((NC, NB, D), lambda p, i: (0, i * (1 - p), 0)),
            pl.BlockSpec((NB, D), lambda p, i: (i * (1 - p), 0)),
            pl.BlockSpec((NB, 16), lambda p, i: (i, 0)),
            pl.BlockSpec((1, D), lambda p, i: (0, 0)),
            pl.BlockSpec((1, D), lambda p, i: (0, 0)),
            pl.BlockSpec((1, D), lambda p, i: (0, 0)),
            pl.BlockSpec((NB, 1), lambda p, i: (i, 0)),
            pl.BlockSpec((D, D), lambda p, i: (0, 0)),
            pl.BlockSpec((1, D), lambda p, i: (0, 0)),
            pl.BlockSpec((D, C), lambda p, i: (0, 0)),
            pl.BlockSpec((1, C), lambda p, i: (0, 0)),
        ],
        out_specs=pl.BlockSpec((G, C), lambda p, i: (0, 0)),
        out_shape=jax.ShapeDtypeStruct((G, C), F32),
        scratch_shapes=[
            pltpu.VMEM((NBLK, NB, D), F32),
            pltpu.VMEM((8, D), F32),
            pltpu.VMEM((G, D), F32),
            pltpu.VMEM((G, 1), F32),
        ],
    )(acc, hp, dinv, b, g, bt, batch2, l1w, l1b, l2w, l2b)


def kernel(x, edge_index, batch, W0, b0, g0, bt0, W1, b1, g1, bt1,
           W2, b2, g2, bt2, l1w, l1b, l2w, l2b):
    src3 = edge_index[0].reshape(NW, NGRP, CPG, K)
    dst3 = edge_index[1].reshape(NW, NGRP, CPG, K)
    ones16 = jnp.concatenate(
        [jnp.ones((K, 1), F32), jnp.zeros((K, 15), F32)], axis=1)
    z16 = jnp.zeros((RPT, 16), F32)
    z128 = jnp.zeros((RPT, D), F32)
    batch2 = batch.reshape(N, 1)

    cnt = _deg_sc(dst3, ones16, z16)
    hp, dinv = _mm0(x, W0, cnt)

    acc = _scatter_sc(hp, src3, dst3, z128)
    hp = _combine_mm(acc, hp, dinv, b0.reshape(1, D), g0.reshape(1, D),
                     bt0.reshape(1, D), W1)
    acc = _scatter_sc(hp, src3, dst3, z128)
    hp = _combine_mm(acc, hp, dinv, b1.reshape(1, D), g1.reshape(1, D),
                     bt1.reshape(1, D), W2)
    acc = _scatter_sc(hp, src3, dst3, z128)
    return _combine_pool(acc, hp, dinv, b2.reshape(1, D), g2.reshape(1, D),
                         bt2.reshape(1, D), batch2, l1w, l1b.reshape(1, D),
                         l2w, l2b.reshape(1, C))


# speedup vs baseline: 24.9854x; 1.0864x over previous
import functools

import jax
import jax.numpy as jnp
from jax import lax
from jax.experimental import pallas as pl
from jax.experimental.pallas import tpu as pltpu
from jax.experimental.pallas import tpu_sc as plsc

N = 10000
D = 128
E = 320000
G = 64
C = 10
EPS = 1e-5

NC, NS = 2, 16
NW = NC * NS
EPW = E // NW
K = 125
NCH = EPW // K
NGRP, CPG = 4, 20
NP = 10240
RPT = NP // NS
NB = 1000
NBLK = N // NB
F32 = jnp.float32


def _sc_mesh():
    return plsc.VectorSubcoreMesh(
        core_axis_name="c", subcore_axis_name="s", num_cores=NC, num_subcores=NS)


def _deg_sc(dst3, ones16, z16):
    @functools.partial(
        pl.kernel,
        out_type=jax.ShapeDtypeStruct((NC, NP, 8), F32),
        mesh=_sc_mesh(),
        scratch_types=[
            pltpu.VMEM((CPG, K), jnp.int32),
            pltpu.VMEM((K, 8), F32),
            pltpu.VMEM_SHARED((NP, 8), F32),
        ],
    )
    def deg_kernel(dst_hbm, ones_hbm, z_hbm, out_hbm, idx_v, ones_v, acc_sh):
        c = lax.axis_index("c")
        s = lax.axis_index("s")
        w = c * NS + s
        pltpu.sync_copy(z_hbm, acc_sh.at[pl.ds(s * RPT, RPT)])
        pltpu.sync_copy(ones_hbm, ones_v)
        plsc.subcore_barrier()

        def group(gi, carry):
            pltpu.sync_copy(dst_hbm.at[w, gi], idx_v)

            def body(j, c2):
                pltpu.sync_copy(ones_v, acc_sh.at[idx_v.at[j]], add=True)
                return c2

            lax.fori_loop(0, CPG, body, 0)
            return carry

        lax.fori_loop(0, NGRP, group, 0)
        plsc.subcore_barrier()
        pltpu.sync_copy(acc_sh.at[pl.ds(s * RPT, RPT)],
                        out_hbm.at[c].at[pl.ds(s * RPT, RPT)])

    return deg_kernel(dst3, ones16, z16)


def _scatter_sc(h, src3, dst3, z128):
    @functools.partial(
        pl.kernel,
        out_type=jax.ShapeDtypeStruct((NC, NP, D), F32),
        mesh=_sc_mesh(),
        scratch_types=[
            pltpu.VMEM((CPG, K), jnp.int32),
            pltpu.VMEM((CPG, K), jnp.int32),
            pltpu.VMEM((K, D), F32),
            pltpu.VMEM((K, D), F32),
            pltpu.VMEM_SHARED((NP, D), F32),
            pltpu.SemaphoreType.DMA,
            pltpu.SemaphoreType.DMA,
        ],
    )
    def scat_kernel(h_hbm, src_hbm, dst_hbm, z_hbm, out_hbm,
                    srcv, dstv, rows_a, rows_b, acc_sh, sem_a, sem_b):
        c = lax.axis_index("c")
        s = lax.axis_index("s")
        w = c * NS + s
        pltpu.sync_copy(z_hbm, acc_sh.at[pl.ds(s * RPT, RPT)])
        plsc.subcore_barrier()

        def g_start(j, buf, sem):
            pltpu.async_copy(h_hbm.at[srcv.at[j]], buf, sem)

        def g_wait(buf, sem):
            pltpu.make_async_copy(h_hbm.at[srcv.at[0]], buf, sem).wait()

        def scat(j, buf):
            pltpu.sync_copy(buf, acc_sh.at[dstv.at[j]], add=True)

        def group(gi, carry):
            pltpu.sync_copy(src_hbm.at[w, gi], srcv)
            pltpu.sync_copy(dst_hbm.at[w, gi], dstv)
            g_start(0, rows_a, sem_a)

            def body(k, c2):
                j = 2 * k
                g_start(j + 1, rows_b, sem_b)
                g_wait(rows_a, sem_a)
                scat(j, rows_a)
                g_start(j + 2, rows_a, sem_a)
                g_wait(rows_b, sem_b)
                scat(j + 1, rows_b)
                return c2

            lax.fori_loop(0, CPG // 2 - 1, body, 0)
            g_start(CPG - 1, rows_b, sem_b)
            g_wait(rows_a, sem_a)
            scat(CPG - 2, rows_a)
            g_wait(rows_b, sem_b)
            scat(CPG - 1, rows_b)
            return carry

        lax.fori_loop(0, NGRP, group, 0)
        plsc.subcore_barrier()
        pltpu.sync_copy(acc_sh.at[pl.ds(s * RPT, RPT)],
                        out_hbm.at[c].at[pl.ds(s * RPT, RPT)])

    return scat_kernel(h, src3, dst3, z128)


def _mm0(x, W, cnt):
    def body(x_ref, w_ref, cnt_ref, hp_ref, dinv_ref):
        deg = cnt_ref[0, :, 0:1] + cnt_ref[1, :, 0:1] + 1.0
        dv = lax.rsqrt(deg)
        h = jnp.dot(x_ref[...], w_ref[...], preferred_element_type=F32)
        hp_ref[...] = h * dv
        dinv_ref[...] = jnp.broadcast_to(dv, (NB, 16))

    return pl.pallas_call(
        body,
        grid=(NBLK,),
        in_specs=[
            pl.BlockSpec((NB, D), lambda i: (i, 0)),
            pl.BlockSpec((D, D), lambda i: (0, 0)),
            pl.BlockSpec((NC, NB, 8), lambda i: (0, i, 0)),
        ],
        out_specs=[
            pl.BlockSpec((NB, D), lambda i: (i, 0)),
            pl.BlockSpec((NB, 16), lambda i: (i, 0)),
        ],
        out_shape=[
            jax.ShapeDtypeStruct((N, D), F32),
            jax.ShapeDtypeStruct((N, 16), F32),
        ],
    )(x, W, cnt)


def _combine_mm(acc, hp, dinv, b, g, bt, W):
    def body(acc_ref, hp_ref, dinv_ref, b_ref, g_ref, bt_ref, w_ref,
             out_ref, t_sc, st_sc):
        p = pl.program_id(0)
        i = pl.program_id(1)

        @pl.when((p == 0) & (i == 0))
        def _():
            st_sc[...] = jnp.zeros_like(st_sc)

        @pl.when(p == 0)
        def _():
            dv = dinv_ref[:, 0:1]
            o = dv * (acc_ref[0] + acc_ref[1] + hp_ref[...]) + b_ref[...]
            t = jnp.where(o > 0, o, jnp.exp(o) - 1.0)
            t_sc[i] = t
            st_sc[0:1, :] += jnp.sum(t, axis=0, keepdims=True)
            st_sc[1:2, :] += jnp.sum(t * t, axis=0, keepdims=True)

        @pl.when(p == 1)
        def _():
            m = st_sc[0:1, :] / N
            v = st_sc[1:2, :] / N - m * m
            sc = g_ref[...] * lax.rsqrt(v + EPS)
            sh = bt_ref[...] - m * sc
            y = t_sc[i] * sc + sh
            h = jnp.dot(y, w_ref[...], preferred_element_type=F32)
            out_ref[...] = h * dinv_ref[:, 0:1]

    return pl.pallas_call(
        body,
        grid=(2, NBLK),
        in_specs=[
            pl.BlockSpec((NC, NB, D), lambda p, i: (0, i * (1 - p), 0)),
            pl.BlockSpec((NB, D), lambda p, i: (i * (1 - p), 0)),
            pl.BlockSpec((NB, 16), lambda p, i: (i, 0)),
            pl.BlockSpec((1, D), lambda p, i: (0, 0)),
            pl.BlockSpec((1, D), lambda p, i: (0, 0)),
            pl.BlockSpec((1, D), lambda p, i: (0, 0)),
            pl.BlockSpec((D, D), lambda p, i: (0, 0)),
        ],
        out_specs=pl.BlockSpec((NB, D), lambda p, i: (i, 0)),
        out_shape=jax.ShapeDtypeStruct((N, D), F32),
        scratch_shapes=[
            pltpu.VMEM((NBLK, NB, D), F32),
            pltpu.VMEM((8, D), F32),
        ],
    )(acc, hp, dinv, b, g, bt, W)


def _combine_pool(acc, hp, dinv, b, g, bt, batch2, l1w, l1b, l2w, l2b):
    def body(acc_ref, hp_ref, dinv_ref, b_ref, g_ref, bt_ref, batch_ref,
             l1w_ref, l1b_ref, l2w_ref, l2b_ref, out_ref,
             t_sc, st_sc, pool_sc, cnt_sc):
        p = pl.program_id(0)
        i = pl.program_id(1)

        @pl.when((p == 0) & (i == 0))
        def _():
            st_sc[...] = jnp.zeros_like(st_sc)
            pool_sc[...] = jnp.zeros_like(pool_sc)
            cnt_sc[...] = jnp.zeros_like(cnt_sc)

        @pl.when(p == 0)
        def _():
            dv = dinv_ref[:, 0:1]
            o = dv * (acc_ref[0] + acc_ref[1] + hp_ref[...]) + b_ref[...]
            t = jnp.where(o > 0, o, jnp.exp(o) - 1.0)
            t_sc[i] = t
            st_sc[0:1, :] += jnp.sum(t, axis=0, keepdims=True)
            st_sc[1:2, :] += jnp.sum(t * t, axis=0, keepdims=True)

        @pl.when(p == 1)
        def _():
            m = st_sc[0:1, :] / N
            v = st_sc[1:2, :] / N - m * m
            sc = g_ref[...] * lax.rsqrt(v + EPS)
            sh = bt_ref[...] - m * sc
            y = t_sc[i] * sc + sh
            gids = lax.broadcasted_iota(jnp.int32, (1, G), 1)
            oh = (batch_ref[...] == gids).astype(F32)
            pool_sc[...] += lax.dot_general(
                oh, y, (((0,), (0,)), ((), ())), preferred_element_type=F32)
            cnt_sc[...] += lax.dot_general(
                oh, jnp.ones((NB, 1), F32), (((0,), (0,)), ((), ())),
                preferred_element_type=F32)

        @pl.when((p == 1) & (i == NBLK - 1))
        def _():
            cnt = jnp.maximum(cnt_sc[...], 1.0)
            pooled = pool_sc[...] / cnt
            z = jnp.dot(pooled, l1w_ref[...], preferred_element_type=F32)
            z = jnp.maximum(z + l1b_ref[...], 0.0)
            z2 = jnp.dot(z, l2w_ref[...], preferred_element_type=F32)
            z2 = z2 + l2b_ref[...]
            mx = jnp.max(z2, axis=-1, keepdims=True)
            lse = jnp.log(jnp.sum(jnp.exp(z2 - mx), axis=-1, keepdims=True)) + mx
            out_ref[...] = z2 - lse

    return pl.pallas_call(
        body,
        grid=(2, NBLK),
        in_specs=[
            pl.BlockSpec((NC, NB, D), lambda p, i: (0, i * (1 - p), 0)),
            pl.BlockSpec((NB, D), lambda p, i: (i * (1 - p), 0)),
            pl.BlockSpec((NB, 16), lambda p, i: (i, 0)),
            pl.BlockSpec((1, D), lambda p, i: (0, 0)),
            pl.BlockSpec((1, D), lambda p, i: (0, 0)),
            pl.BlockSpec((1, D), lambda p, i: (0, 0)),
            pl.BlockSpec((NB, 1), lambda p, i: (i, 0)),
            pl.BlockSpec((D, D), lambda p, i: (0, 0)),
            pl.BlockSpec((1, D), lambda p, i: (0, 0)),
            pl.BlockSpec((D, C), lambda p, i: (0, 0)),
            pl.BlockSpec((1, C), lambda p, i: (0, 0)),
        ],
        out_specs=pl.BlockSpec((G, C), lambda p, i: (0, 0)),
        out_shape=jax.ShapeDtypeStruct((G, C), F32),
        scratch_shapes=[
            pltpu.VMEM((NBLK, NB, D), F32),
            pltpu.VMEM((8, D), F32),
            pltpu.VMEM((G, D), F32),
            pltpu.VMEM((G, 1), F32),
        ],
    )(acc, hp, dinv, b, g, bt, batch2, l1w, l1b, l2w, l2b)


def kernel(x, edge_index, batch, W0, b0, g0, bt0, W1, b1, g1, bt1,
           W2, b2, g2, bt2, l1w, l1b, l2w, l2b):
    src3 = edge_index[0].reshape(NW, NGRP, CPG, K)
    dst3 = edge_index[1].reshape(NW, NGRP, CPG, K)
    ones16 = jnp.concatenate(
        [jnp.ones((K, 1), F32), jnp.zeros((K, 7), F32)], axis=1)
    z16 = jnp.zeros((RPT, 8), F32)
    z128 = jnp.zeros((RPT, D), F32)
    batch2 = batch.reshape(N, 1)

    cnt = _deg_sc(dst3, ones16, z16)
    hp, dinv = _mm0(x, W0, cnt)

    acc = _scatter_sc(hp, src3, dst3, z128)
    hp = _combine_mm(acc, hp, dinv, b0.reshape(1, D), g0.reshape(1, D),
                     bt0.reshape(1, D), W1)
    acc = _scatter_sc(hp, src3, dst3, z128)
    hp = _combine_mm(acc, hp, dinv, b1.reshape(1, D), g1.reshape(1, D),
                     bt1.reshape(1, D), W2)
    acc = _scatter_sc(hp, src3, dst3, z128)
    return _combine_pool(acc, hp, dinv, b2.reshape(1, D), g2.reshape(1, D),
                         bt2.reshape(1, D), batch2, l1w, l1b.reshape(1, D),
                         l2w, l2b.reshape(1, C))


# speedup vs baseline: 26.7574x; 1.0709x over previous
import functools

import jax
import jax.numpy as jnp
from jax import lax
from jax.experimental import pallas as pl
from jax.experimental.pallas import tpu as pltpu
from jax.experimental.pallas import tpu_sc as plsc

N = 10000
D = 128
E = 320000
G = 64
C = 10
EPS = 1e-5

NC, NS = 2, 16
NW = NC * NS
EPW = E // NW
K = 100
NCH = EPW // K
NGRP, CPG = 5, 20
NP = 10240
RPT = NP // NS
NB = 1000
NBLK = N // NB
F32 = jnp.float32


def _sc_mesh():
    return plsc.VectorSubcoreMesh(
        core_axis_name="c", subcore_axis_name="s", num_cores=NC, num_subcores=NS)


def _deg_sc(dst3, ones16, z16):
    @functools.partial(
        pl.kernel,
        out_type=jax.ShapeDtypeStruct((NC, NP, 8), F32),
        mesh=_sc_mesh(),
        scratch_types=[
            pltpu.VMEM((NCH, K), jnp.int32),
            pltpu.VMEM((K, 8), F32),
            pltpu.VMEM_SHARED((NP, 8), F32),
        ],
    )
    def deg_kernel(dst_hbm, ones_hbm, z_hbm, out_hbm, idx_v, ones_v, acc_sh):
        c = lax.axis_index("c")
        s = lax.axis_index("s")
        w = c * NS + s
        pltpu.sync_copy(z_hbm, acc_sh.at[pl.ds(s * RPT, RPT)])
        pltpu.sync_copy(ones_hbm, ones_v)
        pltpu.sync_copy(dst_hbm.at[w], idx_v)
        plsc.subcore_barrier()

        def body(j, c2):
            pltpu.sync_copy(ones_v, acc_sh.at[idx_v.at[j]], add=True)
            return c2

        lax.fori_loop(0, NCH, body, 0)
        plsc.subcore_barrier()
        pltpu.sync_copy(acc_sh.at[pl.ds(s * RPT, RPT)],
                        out_hbm.at[c].at[pl.ds(s * RPT, RPT)])

    return deg_kernel(dst3, ones16, z16)


def _scatter_sc(h, src3, dst3, z128):
    @functools.partial(
        pl.kernel,
        out_type=jax.ShapeDtypeStruct((NC, NP, D), F32),
        mesh=_sc_mesh(),
        scratch_types=[
            pltpu.VMEM((CPG, K), jnp.int32),
            pltpu.VMEM((CPG, K), jnp.int32),
            pltpu.VMEM((K, D), F32),
            pltpu.VMEM((K, D), F32),
            pltpu.VMEM((K, D), F32),
            pltpu.VMEM_SHARED((NP, D), F32),
            pltpu.SemaphoreType.DMA,
            pltpu.SemaphoreType.DMA,
            pltpu.SemaphoreType.DMA,
            pltpu.SemaphoreType.DMA,
        ],
    )
    def scat_kernel(h_hbm, src_hbm, dst_hbm, z_hbm, out_hbm,
                    srcv, dstv, rows_a, rows_b, rows_c, acc_sh,
                    sga, sgb, sgc, sem_s):
        c = lax.axis_index("c")
        s = lax.axis_index("s")
        w = c * NS + s
        pltpu.sync_copy(z_hbm, acc_sh.at[pl.ds(s * RPT, RPT)])
        plsc.subcore_barrier()

        def g_start(j, buf, sem):
            pltpu.async_copy(h_hbm.at[srcv.at[j]], buf, sem)

        def g_wait(buf, sem):
            pltpu.make_async_copy(h_hbm.at[srcv.at[0]], buf, sem).wait()

        def s_start(j, buf):
            pltpu.async_copy(buf, acc_sh.at[dstv.at[j]], sem_s, add=True)

        def s_wait():
            pltpu.make_async_copy(rows_a, acc_sh.at[dstv.at[0]], sem_s).wait()

        bufs = ((rows_a, sga), (rows_b, sgb), (rows_c, sgc))

        def group(gi, carry):
            pltpu.sync_copy(src_hbm.at[w, gi], srcv)
            pltpu.sync_copy(dst_hbm.at[w, gi], dstv)
            g_start(0, rows_a, sga)
            g_start(1, rows_b, sgb)
            g_wait(rows_a, sga)
            s_start(0, rows_a)
            g_start(2, rows_c, sgc)

            def tri(k, c2):
                j0 = 3 * k + 1
                for d in range(3):
                    j = j0 + d
                    buf, sem = bufs[(1 + d) % 3]
                    nbuf, nsem = bufs[d % 3]
                    g_wait(buf, sem)
                    s_start(j, buf)
                    s_wait()
                    if d < 2:
                        g_start(j + 2, nbuf, nsem)
                    else:
                        @pl.when(j + 2 < CPG)
                        def _():
                            g_start(j + 2, nbuf, nsem)

                return c2

            lax.fori_loop(0, (CPG - 2) // 3, tri, 0)
            ebuf, esem = bufs[(CPG - 1) % 3]
            g_wait(ebuf, esem)
            s_start(CPG - 1, ebuf)
            s_wait()
            s_wait()
            return carry

        lax.fori_loop(0, NGRP, group, 0)
        plsc.subcore_barrier()
        pltpu.sync_copy(acc_sh.at[pl.ds(s * RPT, RPT)],
                        out_hbm.at[c].at[pl.ds(s * RPT, RPT)])

    return scat_kernel(h, src3, dst3, z128)


def _mm0(x, W, cnt):
    def body(x_ref, w_ref, cnt_ref, hp_ref, dinv_ref):
        deg = cnt_ref[0, :, 0:1] + cnt_ref[1, :, 0:1] + 1.0
        dv = lax.rsqrt(deg)
        h = jnp.dot(x_ref[...], w_ref[...], preferred_element_type=F32)
        hp_ref[...] = h * dv
        dinv_ref[...] = jnp.broadcast_to(dv, (NB, 16))

    return pl.pallas_call(
        body,
        grid=(NBLK,),
        in_specs=[
            pl.BlockSpec((NB, D), lambda i: (i, 0)),
            pl.BlockSpec((D, D), lambda i: (0, 0)),
            pl.BlockSpec((NC, NB, 8), lambda i: (0, i, 0)),
        ],
        out_specs=[
            pl.BlockSpec((NB, D), lambda i: (i, 0)),
            pl.BlockSpec((NB, 16), lambda i: (i, 0)),
        ],
        out_shape=[
            jax.ShapeDtypeStruct((N, D), F32),
            jax.ShapeDtypeStruct((N, 16), F32),
        ],
    )(x, W, cnt)


def _combine_mm(acc, hp, dinv, b, g, bt, W):
    def body(acc_ref, hp_ref, dinv_ref, b_ref, g_ref, bt_ref, w_ref,
             out_ref, t_sc, st_sc):
        p = pl.program_id(0)
        i = pl.program_id(1)

        @pl.when((p == 0) & (i == 0))
        def _():
            st_sc[...] = jnp.zeros_like(st_sc)

        @pl.when(p == 0)
        def _():
            dv = dinv_ref[:, 0:1]
            o = dv * (acc_ref[0] + acc_ref[1] + hp_ref[...]) + b_ref[...]
            t = jnp.where(o > 0, o, jnp.exp(o) - 1.0)
            t_sc[i] = t
            st_sc[0:1, :] += jnp.sum(t, axis=0, keepdims=True)
            st_sc[1:2, :] += jnp.sum(t * t, axis=0, keepdims=True)

        @pl.when(p == 1)
        def _():
            m = st_sc[0:1, :] / N
            v = st_sc[1:2, :] / N - m * m
            sc = g_ref[...] * lax.rsqrt(v + EPS)
            sh = bt_ref[...] - m * sc
            y = t_sc[i] * sc + sh
            h = jnp.dot(y, w_ref[...], preferred_element_type=F32)
            out_ref[...] = h * dinv_ref[:, 0:1]

    return pl.pallas_call(
        body,
        grid=(2, NBLK),
        in_specs=[
            pl.BlockSpec((NC, NB, D), lambda p, i: (0, i * (1 - p), 0)),
            pl.BlockSpec((NB, D), lambda p, i: (i * (1 - p), 0)),
            pl.BlockSpec((NB, 16), lambda p, i: (i, 0)),
            pl.BlockSpec((1, D), lambda p, i: (0, 0)),
            pl.BlockSpec((1, D), lambda p, i: (0, 0)),
            pl.BlockSpec((1, D), lambda p, i: (0, 0)),
            pl.BlockSpec((D, D), lambda p, i: (0, 0)),
        ],
        out_specs=pl.BlockSpec((NB, D), lambda p, i: (i, 0)),
        out_shape=jax.ShapeDtypeStruct((N, D), F32),
        scratch_shapes=[
            pltpu.VMEM((NBLK, NB, D), F32),
            pltpu.VMEM((8, D), F32),
        ],
    )(acc, hp, dinv, b, g, bt, W)


def _combine_pool(acc, hp, dinv, b, g, bt, batch2, l1w, l1b, l2w, l2b):
    def body(acc_ref, hp_ref, dinv_ref, b_ref, g_ref, bt_ref, batch_ref,
             l1w_ref, l1b_ref, l2w_ref, l2b_ref, out_ref,
             t_sc, st_sc, pool_sc, cnt_sc):
        p = pl.program_id(0)
        i = pl.program_id(1)

        @pl.when((p == 0) & (i == 0))
        def _():
            st_sc[...] = jnp.zeros_like(st_sc)
            pool_sc[...] = jnp.zeros_like(pool_sc)
            cnt_sc[...] = jnp.zeros_like(cnt_sc)

        @pl.when(p == 0)
        def _():
            dv = dinv_ref[:, 0:1]
            o = dv * (acc_ref[0] + acc_ref[1] + hp_ref[...]) + b_ref[...]
            t = jnp.where(o > 0, o, jnp.exp(o) - 1.0)
            t_sc[i] = t
            st_sc[0:1, :] += jnp.sum(t, axis=0, keepdims=True)
            st_sc[1:2, :] += jnp.sum(t * t, axis=0, keepdims=True)

        @pl.when(p == 1)
        def _():
            m = st_sc[0:1, :] / N
            v = st_sc[1:2, :] / N - m * m
            sc = g_ref[...] * lax.rsqrt(v + EPS)
            sh = bt_ref[...] - m * sc
            y = t_sc[i] * sc + sh
            gids = lax.broadcasted_iota(jnp.int32, (1, G), 1)
            oh = (batch_ref[...] == gids).astype(F32)
            pool_sc[...] += lax.dot_general(
                oh, y, (((0,), (0,)), ((), ())), preferred_element_type=F32)
            cnt_sc[...] += lax.dot_general(
                oh, jnp.ones((NB, 1), F32), (((0,), (0,)), ((), ())),
                preferred_element_type=F32)

        @pl.when((p == 1) & (i == NBLK - 1))
        def _():
            cnt = jnp.maximum(cnt_sc[...], 1.0)
            pooled = pool_sc[...] / cnt
            z = jnp.dot(pooled, l1w_ref[...], preferred_element_type=F32)
            z = jnp.maximum(z + l1b_ref[...], 0.0)
            z2 = jnp.dot(z, l2w_ref[...], preferred_element_type=F32)
            z2 = z2 + l2b_ref[...]
            mx = jnp.max(z2, axis=-1, keepdims=True)
            lse = jnp.log(jnp.sum(jnp.exp(z2 - mx), axis=-1, keepdims=True)) + mx
            out_ref[...] = z2 - lse

    return pl.pallas_call(
        body,
        grid=(2, NBLK),
        in_specs=[
            pl.BlockSpec((NC, NB, D), lambda p, i: (0, i * (1 - p), 0)),
            pl.BlockSpec((NB, D), lambda p, i: (i * (1 - p), 0)),
            pl.BlockSpec((NB, 16), lambda p, i: (i, 0)),
            pl.BlockSpec((1, D), lambda p, i: (0, 0)),
            pl.BlockSpec((1, D), lambda p, i: (0, 0)),
            pl.BlockSpec((1, D), lambda p, i: (0, 0)),
            pl.BlockSpec((NB, 1), lambda p, i: (i, 0)),
            pl.BlockSpec((D, D), lambda p, i: (0, 0)),
            pl.BlockSpec((1, D), lambda p, i: (0, 0)),
            pl.BlockSpec((D, C), lambda p, i: (0, 0)),
            pl.BlockSpec((1, C), lambda p, i: (0, 0)),
        ],
        out_specs=pl.BlockSpec((G, C), lambda p, i: (0, 0)),
        out_shape=jax.ShapeDtypeStruct((G, C), F32),
        scratch_shapes=[
            pltpu.VMEM((NBLK, NB, D), F32),
            pltpu.VMEM((8, D), F32),
            pltpu.VMEM((G, D), F32),
            pltpu.VMEM((G, 1), F32),
        ],
    )(acc, hp, dinv, b, g, bt, batch2, l1w, l1b, l2w, l2b)


def kernel(x, edge_index, batch, W0, b0, g0, bt0, W1, b1, g1, bt1,
           W2, b2, g2, bt2, l1w, l1b, l2w, l2b):
    src4 = edge_index[0].reshape(NW, NGRP, CPG, K)
    dst4 = edge_index[1].reshape(NW, NGRP, CPG, K)
    dst3 = edge_index[1].reshape(NW, NCH, K)
    ones16 = jnp.concatenate(
        [jnp.ones((K, 1), F32), jnp.zeros((K, 7), F32)], axis=1)
    z16 = jnp.zeros((RPT, 8), F32)
    z128 = jnp.zeros((RPT, D), F32)
    batch2 = batch.reshape(N, 1)

    cnt = _deg_sc(dst3, ones16, z16)
    hp, dinv = _mm0(x, W0, cnt)

    acc = _scatter_sc(hp, src4, dst4, z128)
    hp = _combine_mm(acc, hp, dinv, b0.reshape(1, D), g0.reshape(1, D),
                     bt0.reshape(1, D), W1)
    acc = _scatter_sc(hp, src4, dst4, z128)
    hp = _combine_mm(acc, hp, dinv, b1.reshape(1, D), g1.reshape(1, D),
                     bt1.reshape(1, D), W2)
    acc = _scatter_sc(hp, src4, dst4, z128)
    return _combine_pool(acc, hp, dinv, b2.reshape(1, D), g2.reshape(1, D),
                         bt2.reshape(1, D), batch2, l1w, l1b.reshape(1, D),
                         l2w, l2b.reshape(1, C))


# speedup vs baseline: 28.0964x; 1.0500x over previous
import functools

import jax
import jax.numpy as jnp
from jax import lax
from jax.experimental import pallas as pl
from jax.experimental.pallas import tpu as pltpu
from jax.experimental.pallas import tpu_sc as plsc

N = 10000
D = 128
E = 320000
G = 64
C = 10
EPS = 1e-5

NC, NS = 2, 16
NW = NC * NS
EPW = E // NW
K = 100
NCH = EPW // K
NGRP, CPG = 5, 20
NP = 10240
RPT = NP // NS
NB = 2000
NBLK = N // NB
F32 = jnp.float32


def _sc_mesh():
    return plsc.VectorSubcoreMesh(
        core_axis_name="c", subcore_axis_name="s", num_cores=NC, num_subcores=NS)


def _deg_sc(dst3, ones16, z16):
    @functools.partial(
        pl.kernel,
        out_type=jax.ShapeDtypeStruct((NC, NP, 8), F32),
        mesh=_sc_mesh(),
        scratch_types=[
            pltpu.VMEM((NCH, K), jnp.int32),
            pltpu.VMEM((K, 8), F32),
            pltpu.VMEM_SHARED((NP, 8), F32),
            pltpu.SemaphoreType.DMA,
        ],
    )
    def deg_kernel(dst_hbm, ones_hbm, z_hbm, out_hbm, idx_v, ones_v, acc_sh,
                   sem):
        c = lax.axis_index("c")
        s = lax.axis_index("s")
        w = c * NS + s
        pltpu.sync_copy(z_hbm, acc_sh.at[pl.ds(s * RPT, RPT)])
        pltpu.sync_copy(ones_hbm, ones_v)
        pltpu.sync_copy(dst_hbm.at[w], idx_v)
        plsc.subcore_barrier()

        def fire(j, c2):
            pltpu.async_copy(ones_v, acc_sh.at[idx_v.at[j]], sem, add=True)
            return c2

        lax.fori_loop(0, NCH, fire, 0)

        def drain(j, c2):
            pltpu.make_async_copy(ones_v, acc_sh.at[idx_v.at[0]], sem).wait()
            return c2

        lax.fori_loop(0, NCH, drain, 0)
        plsc.subcore_barrier()
        pltpu.sync_copy(acc_sh.at[pl.ds(s * RPT, RPT)],
                        out_hbm.at[c].at[pl.ds(s * RPT, RPT)])

    return deg_kernel(dst3, ones16, z16)


def _scatter_sc(h, src3, dst3, z128):
    @functools.partial(
        pl.kernel,
        out_type=jax.ShapeDtypeStruct((NC, NP, D), F32),
        mesh=_sc_mesh(),
        scratch_types=[
            pltpu.VMEM((CPG, K), jnp.int32),
            pltpu.VMEM((CPG, K), jnp.int32),
            pltpu.VMEM((K, D), F32),
            pltpu.VMEM((K, D), F32),
            pltpu.VMEM((K, D), F32),
            pltpu.VMEM_SHARED((NP, D), F32),
            pltpu.SemaphoreType.DMA,
            pltpu.SemaphoreType.DMA,
            pltpu.SemaphoreType.DMA,
            pltpu.SemaphoreType.DMA,
        ],
    )
    def scat_kernel(h_hbm, src_hbm, dst_hbm, z_hbm, out_hbm,
                    srcv, dstv, rows_a, rows_b, rows_c, acc_sh,
                    sga, sgb, sgc, sem_s):
        c = lax.axis_index("c")
        s = lax.axis_index("s")
        w = c * NS + s
        pltpu.sync_copy(z_hbm, acc_sh.at[pl.ds(s * RPT, RPT)])
        plsc.subcore_barrier()

        def g_start(j, buf, sem):
            pltpu.async_copy(h_hbm.at[srcv.at[j]], buf, sem)

        def g_wait(buf, sem):
            pltpu.make_async_copy(h_hbm.at[srcv.at[0]], buf, sem).wait()

        def s_start(j, buf):
            pltpu.async_copy(buf, acc_sh.at[dstv.at[j]], sem_s, add=True)

        def s_wait():
            pltpu.make_async_copy(rows_a, acc_sh.at[dstv.at[0]], sem_s).wait()

        bufs = ((rows_a, sga), (rows_b, sgb), (rows_c, sgc))

        def group(gi, carry):
            pltpu.sync_copy(src_hbm.at[w, gi], srcv)
            pltpu.sync_copy(dst_hbm.at[w, gi], dstv)
            g_start(0, rows_a, sga)
            g_start(1, rows_b, sgb)
            g_wait(rows_a, sga)
            s_start(0, rows_a)
            g_start(2, rows_c, sgc)

            def tri(k, c2):
                j0 = 3 * k + 1
                for d in range(3):
                    j = j0 + d
                    buf, sem = bufs[(1 + d) % 3]
                    nbuf, nsem = bufs[d % 3]
                    g_wait(buf, sem)
                    s_start(j, buf)
                    s_wait()
                    if d < 2:
                        g_start(j + 2, nbuf, nsem)
                    else:
                        @pl.when(j + 2 < CPG)
                        def _():
                            g_start(j + 2, nbuf, nsem)

                return c2

            lax.fori_loop(0, (CPG - 2) // 3, tri, 0)
            ebuf, esem = bufs[(CPG - 1) % 3]
            g_wait(ebuf, esem)
            s_start(CPG - 1, ebuf)
            s_wait()
            s_wait()
            return carry

        lax.fori_loop(0, NGRP, group, 0)
        plsc.subcore_barrier()
        pltpu.sync_copy(acc_sh.at[pl.ds(s * RPT, RPT)],
                        out_hbm.at[c].at[pl.ds(s * RPT, RPT)])

    return scat_kernel(h, src3, dst3, z128)


def _mm0(x, W, cnt):
    def body(x_ref, w_ref, cnt_ref, hp_ref, dinv_ref):
        deg = cnt_ref[0, :, 0:1] + cnt_ref[1, :, 0:1] + 1.0
        dv = lax.rsqrt(deg)
        h = jnp.dot(x_ref[...], w_ref[...], preferred_element_type=F32)
        hp_ref[...] = h * dv
        dinv_ref[...] = jnp.broadcast_to(dv, (NB, 16))

    return pl.pallas_call(
        body,
        grid=(NBLK,),
        in_specs=[
            pl.BlockSpec((NB, D), lambda i: (i, 0)),
            pl.BlockSpec((D, D), lambda i: (0, 0)),
            pl.BlockSpec((NC, NB, 8), lambda i: (0, i, 0)),
        ],
        out_specs=[
            pl.BlockSpec((NB, D), lambda i: (i, 0)),
            pl.BlockSpec((NB, 16), lambda i: (i, 0)),
        ],
        out_shape=[
            jax.ShapeDtypeStruct((N, D), F32),
            jax.ShapeDtypeStruct((N, 16), F32),
        ],
    )(x, W, cnt)


def _combine_mm(acc, hp, dinv, b, g, bt, W):
    def body(acc_ref, hp_ref, dinv_ref, b_ref, g_ref, bt_ref, w_ref,
             out_ref, t_sc, st_sc):
        p = pl.program_id(0)
        i = pl.program_id(1)

        @pl.when((p == 0) & (i == 0))
        def _():
            st_sc[...] = jnp.zeros_like(st_sc)

        @pl.when(p == 0)
        def _():
            dv = dinv_ref[:, 0:1]
            o = dv * (acc_ref[0] + acc_ref[1] + hp_ref[...]) + b_ref[...]
            t = jnp.where(o > 0, o, jnp.exp(o) - 1.0)
            t_sc[i] = t
            st_sc[0:1, :] += jnp.sum(t, axis=0, keepdims=True)
            st_sc[1:2, :] += jnp.sum(t * t, axis=0, keepdims=True)

        @pl.when(p == 1)
        def _():
            m = st_sc[0:1, :] / N
            v = st_sc[1:2, :] / N - m * m
            sc = g_ref[...] * lax.rsqrt(v + EPS)
            sh = bt_ref[...] - m * sc
            y = t_sc[i] * sc + sh
            h = jnp.dot(y, w_ref[...], preferred_element_type=F32)
            out_ref[...] = h * dinv_ref[:, 0:1]

    return pl.pallas_call(
        body,
        grid=(2, NBLK),
        in_specs=[
            pl.BlockSpec((NC, NB, D), lambda p, i: (0, i * (1 - p), 0)),
            pl.BlockSpec((NB, D), lambda p, i: (i * (1 - p), 0)),
            pl.BlockSpec((NB, 16), lambda p, i: (i, 0)),
            pl.BlockSpec((1, D), lambda p, i: (0, 0)),
            pl.BlockSpec((1, D), lambda p, i: (0, 0)),
            pl.BlockSpec((1, D), lambda p, i: (0, 0)),
            pl.BlockSpec((D, D), lambda p, i: (0, 0)),
        ],
        out_specs=pl.BlockSpec((NB, D), lambda p, i: (i, 0)),
        out_shape=jax.ShapeDtypeStruct((N, D), F32),
        scratch_shapes=[
            pltpu.VMEM((NBLK, NB, D), F32),
            pltpu.VMEM((8, D), F32),
        ],
    )(acc, hp, dinv, b, g, bt, W)


def _combine_pool(acc, hp, dinv, b, g, bt, batch2, l1w, l1b, l2w, l2b):
    def body(acc_ref, hp_ref, dinv_ref, b_ref, g_ref, bt_ref, batch_ref,
             l1w_ref, l1b_ref, l2w_ref, l2b_ref, out_ref,
             t_sc, st_sc, pool_sc, cnt_sc):
        p = pl.program_id(0)
        i = pl.program_id(1)

        @pl.when((p == 0) & (i == 0))
        def _():
            st_sc[...] = jnp.zeros_like(st_sc)
            pool_sc[...] = jnp.zeros_like(pool_sc)
            cnt_sc[...] = jnp.zeros_like(cnt_sc)

        @pl.when(p == 0)
        def _():
            dv = dinv_ref[:, 0:1]
            o = dv * (acc_ref[0] + acc_ref[1] + hp_ref[...]) + b_ref[...]
            t = jnp.where(o > 0, o, jnp.exp(o) - 1.0)
            t_sc[i] = t
            st_sc[0:1, :] += jnp.sum(t, axis=0, keepdims=True)
            st_sc[1:2, :] += jnp.sum(t * t, axis=0, keepdims=True)

        @pl.when(p == 1)
        def _():
            m = st_sc[0:1, :] / N
            v = st_sc[1:2, :] / N - m * m
            sc = g_ref[...] * lax.rsqrt(v + EPS)
            sh = bt_ref[...] - m * sc
            y = t_sc[i] * sc + sh
            gids = lax.broadcasted_iota(jnp.int32, (1, G), 1)
            oh = (batch_ref[...] == gids).astype(F32)
            pool_sc[...] += lax.dot_general(
                oh, y, (((0,), (0,)), ((), ())), preferred_element_type=F32)
            cnt_sc[...] += lax.dot_general(
                oh, jnp.ones((NB, 1), F32), (((0,), (0,)), ((), ())),
                preferred_element_type=F32)

        @pl.when((p == 1) & (i == NBLK - 1))
        def _():
            cnt = jnp.maximum(cnt_sc[...], 1.0)
            pooled = pool_sc[...] / cnt
            z = jnp.dot(pooled, l1w_ref[...], preferred_element_type=F32)
            z = jnp.maximum(z + l1b_ref[...], 0.0)
            z2 = jnp.dot(z, l2w_ref[...], preferred_element_type=F32)
            z2 = z2 + l2b_ref[...]
            mx = jnp.max(z2, axis=-1, keepdims=True)
            lse = jnp.log(jnp.sum(jnp.exp(z2 - mx), axis=-1, keepdims=True)) + mx
            out_ref[...] = z2 - lse

    return pl.pallas_call(
        body,
        grid=(2, NBLK),
        in_specs=[
            pl.BlockSpec((NC, NB, D), lambda p, i: (0, i * (1 - p), 0)),
            pl.BlockSpec((NB, D), lambda p, i: (i * (1 - p), 0)),
            pl.BlockSpec((NB, 16), lambda p, i: (i, 0)),
            pl.BlockSpec((1, D), lambda p, i: (0, 0)),
            pl.BlockSpec((1, D), lambda p, i: (0, 0)),
            pl.BlockSpec((1, D), lambda p, i: (0, 0)),
            pl.BlockSpec((NB, 1), lambda p, i: (i, 0)),
            pl.BlockSpec((D, D), lambda p, i: (0, 0)),
            pl.BlockSpec((1, D), lambda p, i: (0, 0)),
            pl.BlockSpec((D, C), lambda p, i: (0, 0)),
            pl.BlockSpec((1, C), lambda p, i: (0, 0)),
        ],
        out_specs=pl.BlockSpec((G, C), lambda p, i: (0, 0)),
        out_shape=jax.ShapeDtypeStruct((G, C), F32),
        scratch_shapes=[
            pltpu.VMEM((NBLK, NB, D), F32),
            pltpu.VMEM((8, D), F32),
            pltpu.VMEM((G, D), F32),
            pltpu.VMEM((G, 1), F32),
        ],
    )(acc, hp, dinv, b, g, bt, batch2, l1w, l1b, l2w, l2b)


def kernel(x, edge_index, batch, W0, b0, g0, bt0, W1, b1, g1, bt1,
           W2, b2, g2, bt2, l1w, l1b, l2w, l2b):
    src4 = edge_index[0].reshape(NW, NGRP, CPG, K)
    dst4 = edge_index[1].reshape(NW, NGRP, CPG, K)
    dst3 = edge_index[1].reshape(NW, NCH, K)
    ones16 = jnp.concatenate(
        [jnp.ones((K, 1), F32), jnp.zeros((K, 7), F32)], axis=1)
    z16 = jnp.zeros((RPT, 8), F32)
    z128 = jnp.zeros((RPT, D), F32)
    batch2 = batch.reshape(N, 1)

    cnt = _deg_sc(dst3, ones16, z16)
    hp, dinv = _mm0(x, W0, cnt)

    acc = _scatter_sc(hp, src4, dst4, z128)
    hp = _combine_mm(acc, hp, dinv, b0.reshape(1, D), g0.reshape(1, D),
                     bt0.reshape(1, D), W1)
    acc = _scatter_sc(hp, src4, dst4, z128)
    hp = _combine_mm(acc, hp, dinv, b1.reshape(1, D), g1.reshape(1, D),
                     bt1.reshape(1, D), W2)
    acc = _scatter_sc(hp, src4, dst4, z128)
    return _combine_pool(acc, hp, dinv, b2.reshape(1, D), g2.reshape(1, D),
                         bt2.reshape(1, D), batch2, l1w, l1b.reshape(1, D),
                         l2w, l2b.reshape(1, C))


# speedup vs baseline: 28.8381x; 1.0264x over previous
import functools

import jax
import jax.numpy as jnp
from jax import lax
from jax.experimental import pallas as pl
from jax.experimental.pallas import tpu as pltpu
from jax.experimental.pallas import tpu_sc as plsc

N = 10000
D = 128
E = 320000
G = 64
C = 10
EPS = 1e-5

NC, NS = 2, 16
NW = NC * NS
EPW = E // NW
K = 100
NCH = EPW // K
NGRP, CPG = 5, 20
NP = 10240
RPT = NP // NS
NB = 2000
NBLK = N // NB
F32 = jnp.float32


def _sc_mesh():
    return plsc.VectorSubcoreMesh(
        core_axis_name="c", subcore_axis_name="s", num_cores=NC, num_subcores=NS)


def _deg_sc(ei5, ones8, z8):
    @functools.partial(
        pl.kernel,
        out_type=jax.ShapeDtypeStruct((NC, NP, 8), F32),
        mesh=_sc_mesh(),
        scratch_types=[
            pltpu.VMEM((NGRP, CPG, K), jnp.int32),
            pltpu.VMEM((K, 8), F32),
            pltpu.VMEM_SHARED((NP, 8), F32),
            pltpu.SemaphoreType.DMA,
        ],
    )
    def deg_kernel(ei_hbm, ones_hbm, z_hbm, out_hbm, idx_v, ones_v, acc_sh,
                   sem):
        c = lax.axis_index("c")
        s = lax.axis_index("s")
        w = c * NS + s
        pltpu.sync_copy(z_hbm, acc_sh.at[pl.ds(s * RPT, RPT)])
        pltpu.sync_copy(ones_hbm, ones_v)
        for gi in range(NGRP):
            pltpu.sync_copy(ei_hbm.at[1, w, gi], idx_v.at[gi])
        plsc.subcore_barrier()

        def fire(gi, carry):
            def fire1(j, c2):
                pltpu.async_copy(ones_v, acc_sh.at[idx_v.at[gi, j]], sem,
                                 add=True)
                return c2

            lax.fori_loop(0, CPG, fire1, 0)
            return carry

        lax.fori_loop(0, NGRP, fire, 0)

        def drain(j, c2):
            pltpu.make_async_copy(ones_v, acc_sh.at[idx_v.at[0, 0]], sem).wait()
            return c2

        lax.fori_loop(0, NCH, drain, 0)
        plsc.subcore_barrier()
        pltpu.sync_copy(acc_sh.at[pl.ds(s * RPT, RPT)],
                        out_hbm.at[c].at[pl.ds(s * RPT, RPT)])

    return deg_kernel(ei5, ones8, z8)


def _scatter_sc(h, ei5, z128):
    @functools.partial(
        pl.kernel,
        out_type=jax.ShapeDtypeStruct((NC, NP, D), F32),
        mesh=_sc_mesh(),
        scratch_types=[
            pltpu.VMEM((CPG, K), jnp.int32),
            pltpu.VMEM((CPG, K), jnp.int32),
            pltpu.VMEM((K, D), F32),
            pltpu.VMEM((K, D), F32),
            pltpu.VMEM((K, D), F32),
            pltpu.VMEM_SHARED((NP, D), F32),
            pltpu.SemaphoreType.DMA,
            pltpu.SemaphoreType.DMA,
            pltpu.SemaphoreType.DMA,
            pltpu.SemaphoreType.DMA,
        ],
    )
    def scat_kernel(h_hbm, ei_hbm, z_hbm, out_hbm,
                    srcv, dstv, rows_a, rows_b, rows_c, acc_sh,
                    sga, sgb, sgc, sem_s):
        c = lax.axis_index("c")
        s = lax.axis_index("s")
        w = c * NS + s
        pltpu.sync_copy(z_hbm, acc_sh.at[pl.ds(s * RPT, RPT)])
        plsc.subcore_barrier()

        def g_start(j, buf, sem):
            pltpu.async_copy(h_hbm.at[srcv.at[j]], buf, sem)

        def g_wait(buf, sem):
            pltpu.make_async_copy(h_hbm.at[srcv.at[0]], buf, sem).wait()

        def s_start(j, buf):
            pltpu.async_copy(buf, acc_sh.at[dstv.at[j]], sem_s, add=True)

        def s_wait():
            pltpu.make_async_copy(rows_a, acc_sh.at[dstv.at[0]], sem_s).wait()

        bufs = ((rows_a, sga), (rows_b, sgb), (rows_c, sgc))

        def group(gi, carry):
            pltpu.sync_copy(ei_hbm.at[0, w, gi], srcv)
            pltpu.sync_copy(ei_hbm.at[1, w, gi], dstv)
            g_start(0, rows_a, sga)
            g_start(1, rows_b, sgb)
            g_wait(rows_a, sga)
            s_start(0, rows_a)
            g_start(2, rows_c, sgc)

            def tri(k, c2):
                j0 = 3 * k + 1
                for d in range(3):
                    j = j0 + d
                    buf, sem = bufs[(1 + d) % 3]
                    nbuf, nsem = bufs[d % 3]
                    g_wait(buf, sem)
                    s_start(j, buf)
                    s_wait()
                    if d < 2:
                        g_start(j + 2, nbuf, nsem)
                    else:
                        @pl.when(j + 2 < CPG)
                        def _():
                            g_start(j + 2, nbuf, nsem)

                return c2

            lax.fori_loop(0, (CPG - 2) // 3, tri, 0)
            ebuf, esem = bufs[(CPG - 1) % 3]
            g_wait(ebuf, esem)
            s_start(CPG - 1, ebuf)
            s_wait()
            s_wait()
            return carry

        lax.fori_loop(0, NGRP, group, 0)
        plsc.subcore_barrier()
        pltpu.sync_copy(acc_sh.at[pl.ds(s * RPT, RPT)],
                        out_hbm.at[c].at[pl.ds(s * RPT, RPT)])

    return scat_kernel(h, ei5, z128)


def _mm0(x, W, cnt):
    def body(x_ref, w_ref, cnt_ref, hp_ref, dinv_ref):
        deg = cnt_ref[0, :, 0:1] + cnt_ref[1, :, 0:1] + 1.0
        dv = lax.rsqrt(deg)
        h = jnp.dot(x_ref[...], w_ref[...], preferred_element_type=F32)
        hp_ref[...] = h * dv
        dinv_ref[...] = jnp.broadcast_to(dv, (NB, 16))

    return pl.pallas_call(
        body,
        grid=(NBLK,),
        in_specs=[
            pl.BlockSpec((NB, D), lambda i: (i, 0)),
            pl.BlockSpec((D, D), lambda i: (0, 0)),
            pl.BlockSpec((NC, NB, 8), lambda i: (0, i, 0)),
        ],
        out_specs=[
            pl.BlockSpec((NB, D), lambda i: (i, 0)),
            pl.BlockSpec((NB, 16), lambda i: (i, 0)),
        ],
        out_shape=[
            jax.ShapeDtypeStruct((N, D), F32),
            jax.ShapeDtypeStruct((N, 16), F32),
        ],
    )(x, W, cnt)


def _combine_mm(acc, hp, dinv, b, g, bt, W):
    def body(acc_ref, hp_ref, dinv_ref, b_ref, g_ref, bt_ref, w_ref,
             out_ref, t_sc, st_sc):
        p = pl.program_id(0)
        i = pl.program_id(1)

        @pl.when((p == 0) & (i == 0))
        def _():
            st_sc[...] = jnp.zeros_like(st_sc)

        @pl.when(p == 0)
        def _():
            dv = dinv_ref[:, 0:1]
            o = dv * (acc_ref[0] + acc_ref[1] + hp_ref[...]) + b_ref[...]
            t = jnp.where(o > 0, o, jnp.exp(o) - 1.0)
            t_sc[i] = t
            st_sc[0:1, :] += jnp.sum(t, axis=0, keepdims=True)
            st_sc[1:2, :] += jnp.sum(t * t, axis=0, keepdims=True)

        @pl.when(p == 1)
        def _():
            m = st_sc[0:1, :] / N
            v = st_sc[1:2, :] / N - m * m
            sc = g_ref[...] * lax.rsqrt(v + EPS)
            sh = bt_ref[...] - m * sc
            y = t_sc[i] * sc + sh
            h = jnp.dot(y, w_ref[...], preferred_element_type=F32)
            out_ref[...] = h * dinv_ref[:, 0:1]

    return pl.pallas_call(
        body,
        grid=(2, NBLK),
        in_specs=[
            pl.BlockSpec((NC, NB, D), lambda p, i: (0, i * (1 - p), 0)),
            pl.BlockSpec((NB, D), lambda p, i: (i * (1 - p), 0)),
            pl.BlockSpec((NB, 16), lambda p, i: (i, 0)),
            pl.BlockSpec((1, D), lambda p, i: (0, 0)),
            pl.BlockSpec((1, D), lambda p, i: (0, 0)),
            pl.BlockSpec((1, D), lambda p, i: (0, 0)),
            pl.BlockSpec((D, D), lambda p, i: (0, 0)),
        ],
        out_specs=pl.BlockSpec((NB, D), lambda p, i: (i, 0)),
        out_shape=jax.ShapeDtypeStruct((N, D), F32),
        scratch_shapes=[
            pltpu.VMEM((NBLK, NB, D), F32),
            pltpu.VMEM((8, D), F32),
        ],
    )(acc, hp, dinv, b, g, bt, W)


def _combine_pool(acc, hp, dinv, b, g, bt, batch2, l1w, l1b, l2w, l2b):
    def body(acc_ref, hp_ref, dinv_ref, b_ref, g_ref, bt_ref, batch_ref,
             l1w_ref, l1b_ref, l2w_ref, l2b_ref, out_ref,
             t_sc, st_sc, pool_sc, cnt_sc):
        p = pl.program_id(0)
        i = pl.program_id(1)

        @pl.when((p == 0) & (i == 0))
        def _():
            st_sc[...] = jnp.zeros_like(st_sc)
            pool_sc[...] = jnp.zeros_like(pool_sc)
            cnt_sc[...] = jnp.zeros_like(cnt_sc)

        @pl.when(p == 0)
        def _():
            dv = dinv_ref[:, 0:1]
            o = dv * (acc_ref[0] + acc_ref[1] + hp_ref[...]) + b_ref[...]
            t = jnp.where(o > 0, o, jnp.exp(o) - 1.0)
            t_sc[i] = t
            st_sc[0:1, :] += jnp.sum(t, axis=0, keepdims=True)
            st_sc[1:2, :] += jnp.sum(t * t, axis=0, keepdims=True)

        @pl.when(p == 1)
        def _():
            m = st_sc[0:1, :] / N
            v = st_sc[1:2, :] / N - m * m
            sc = g_ref[...] * lax.rsqrt(v + EPS)
            sh = bt_ref[...] - m * sc
            y = t_sc[i] * sc + sh
            gids = lax.broadcasted_iota(jnp.int32, (G, NB), 0)
            bb = batch_ref[...].reshape(1, NB)
            oh = (bb == gids).astype(F32)
            pool_sc[...] += lax.dot_general(
                oh, y, (((1,), (0,)), ((), ())), preferred_element_type=F32)
            cnt_sc[...] += lax.dot_general(
                oh, jnp.ones((NB, 1), F32), (((1,), (0,)), ((), ())),
                preferred_element_type=F32)

        @pl.when((p == 1) & (i == NBLK - 1))
        def _():
            cnt = jnp.maximum(cnt_sc[...], 1.0)
            pooled = pool_sc[...] / cnt
            z = jnp.dot(pooled, l1w_ref[...], preferred_element_type=F32)
            z = jnp.maximum(z + l1b_ref[...], 0.0)
            z2 = jnp.dot(z, l2w_ref[...], preferred_element_type=F32)
            z2 = z2 + l2b_ref[...]
            mx = jnp.max(z2, axis=-1, keepdims=True)
            lse = jnp.log(jnp.sum(jnp.exp(z2 - mx), axis=-1, keepdims=True)) + mx
            out_ref[...] = z2 - lse

    return pl.pallas_call(
        body,
        grid=(2, NBLK),
        in_specs=[
            pl.BlockSpec((NC, NB, D), lambda p, i: (0, i * (1 - p), 0)),
            pl.BlockSpec((NB, D), lambda p, i: (i * (1 - p), 0)),
            pl.BlockSpec((NB, 16), lambda p, i: (i, 0)),
            pl.BlockSpec((1, D), lambda p, i: (0, 0)),
            pl.BlockSpec((1, D), lambda p, i: (0, 0)),
            pl.BlockSpec((1, D), lambda p, i: (0, 0)),
            pl.BlockSpec((1, 1, NB), lambda p, i: (i, 0, 0)),
            pl.BlockSpec((D, D), lambda p, i: (0, 0)),
            pl.BlockSpec((1, D), lambda p, i: (0, 0)),
            pl.BlockSpec((D, C), lambda p, i: (0, 0)),
            pl.BlockSpec((1, C), lambda p, i: (0, 0)),
        ],
        out_specs=pl.BlockSpec((G, C), lambda p, i: (0, 0)),
        out_shape=jax.ShapeDtypeStruct((G, C), F32),
        scratch_shapes=[
            pltpu.VMEM((NBLK, NB, D), F32),
            pltpu.VMEM((8, D), F32),
            pltpu.VMEM((G, D), F32),
            pltpu.VMEM((G, 1), F32),
        ],
    )(acc, hp, dinv, b, g, bt, batch2, l1w, l1b, l2w, l2b)


def kernel(x, edge_index, batch, W0, b0, g0, bt0, W1, b1, g1, bt1,
           W2, b2, g2, bt2, l1w, l1b, l2w, l2b):
    ei5 = edge_index.reshape(2, NW, NGRP, CPG, K)
    ones16 = jnp.concatenate(
        [jnp.ones((K, 1), F32), jnp.zeros((K, 7), F32)], axis=1)
    z16 = jnp.zeros((RPT, 8), F32)
    z128 = jnp.zeros((RPT, D), F32)
    cnt = _deg_sc(ei5, ones16, z16)
    hp, dinv = _mm0(x, W0, cnt)

    acc = _scatter_sc(hp, ei5, z128)
    hp = _combine_mm(acc, hp, dinv, b0.reshape(1, D), g0.reshape(1, D),
                     bt0.reshape(1, D), W1)
    acc = _scatter_sc(hp, ei5, z128)
    hp = _combine_mm(acc, hp, dinv, b1.reshape(1, D), g1.reshape(1, D),
                     bt1.reshape(1, D), W2)
    acc = _scatter_sc(hp, ei5, z128)
    return _combine_pool(acc, hp, dinv, b2.reshape(1, D), g2.reshape(1, D),
                         bt2.reshape(1, D), batch.reshape(NBLK, 1, NB), l1w, l1b.reshape(1, D),
                         l2w, l2b.reshape(1, C))


# speedup vs baseline: 28.9345x; 1.0033x over previous
import functools

import jax
import jax.numpy as jnp
from jax import lax
from jax.experimental import pallas as pl
from jax.experimental.pallas import tpu as pltpu
from jax.experimental.pallas import tpu_sc as plsc

N = 10000
D = 128
E = 320000
G = 64
C = 10
EPS = 1e-5

NC, NS = 2, 16
NW = NC * NS
EPW = E // NW
K = 100
NCH = EPW // K
NGRP, CPG = 5, 20
NP = 10240
RPT = NP // NS
NB = 2000
NBLK = N // NB
F32 = jnp.float32


def _sc_mesh():
    return plsc.VectorSubcoreMesh(
        core_axis_name="c", subcore_axis_name="s", num_cores=NC, num_subcores=NS)


def _deg_sc(ei5, ones8, z8):
    @functools.partial(
        pl.kernel,
        out_type=jax.ShapeDtypeStruct((NC, NP, 8), F32),
        mesh=_sc_mesh(),
        scratch_types=[
            pltpu.VMEM((NGRP, CPG, K), jnp.int32),
            pltpu.VMEM((K, 8), F32),
            pltpu.VMEM_SHARED((NP, 8), F32),
            pltpu.SemaphoreType.DMA,
        ],
    )
    def deg_kernel(ei_hbm, ones_hbm, z_hbm, out_hbm, idx_v, ones_v, acc_sh,
                   sem):
        c = lax.axis_index("c")
        s = lax.axis_index("s")
        w = c * NS + s
        pltpu.sync_copy(z_hbm, acc_sh.at[pl.ds(s * RPT, RPT)])
        pltpu.sync_copy(ones_hbm, ones_v)
        pltpu.sync_copy(ei_hbm.at[1, w], idx_v)
        plsc.subcore_barrier()

        def fire(gi, carry):
            def fire1(j, c2):
                pltpu.async_copy(ones_v, acc_sh.at[idx_v.at[gi, j]], sem,
                                 add=True)
                return c2

            lax.fori_loop(0, CPG, fire1, 0)
            return carry

        lax.fori_loop(0, NGRP, fire, 0)

        def drain(j, c2):
            pltpu.make_async_copy(ones_v, acc_sh.at[idx_v.at[0, 0]], sem).wait()
            return c2

        lax.fori_loop(0, NCH, drain, 0)
        plsc.subcore_barrier()
        pltpu.sync_copy(acc_sh.at[pl.ds(s * RPT, RPT)],
                        out_hbm.at[c].at[pl.ds(s * RPT, RPT)])

    return deg_kernel(ei5, ones8, z8)


def _scatter_sc(h, ei5, z128):
    @functools.partial(
        pl.kernel,
        out_type=jax.ShapeDtypeStruct((NC, NP, D), F32),
        mesh=_sc_mesh(),
        scratch_types=[
            pltpu.VMEM((CPG, K), jnp.int32),
            pltpu.VMEM((CPG, K), jnp.int32),
            pltpu.VMEM((K, D), F32),
            pltpu.VMEM((K, D), F32),
            pltpu.VMEM((K, D), F32),
            pltpu.VMEM_SHARED((NP, D), F32),
            pltpu.SemaphoreType.DMA,
            pltpu.SemaphoreType.DMA,
            pltpu.SemaphoreType.DMA,
            pltpu.SemaphoreType.DMA,
        ],
    )
    def scat_kernel(h_hbm, ei_hbm, z_hbm, out_hbm,
                    srcv, dstv, rows_a, rows_b, rows_c, acc_sh,
                    sga, sgb, sgc, sem_s):
        c = lax.axis_index("c")
        s = lax.axis_index("s")
        w = c * NS + s
        pltpu.sync_copy(z_hbm, acc_sh.at[pl.ds(s * RPT, RPT)])
        plsc.subcore_barrier()

        def g_start(j, buf, sem):
            pltpu.async_copy(h_hbm.at[srcv.at[j]], buf, sem)

        def g_wait(buf, sem):
            pltpu.make_async_copy(h_hbm.at[srcv.at[0]], buf, sem).wait()

        def s_start(j, buf):
            pltpu.async_copy(buf, acc_sh.at[dstv.at[j]], sem_s, add=True)

        def s_wait():
            pltpu.make_async_copy(rows_a, acc_sh.at[dstv.at[0]], sem_s).wait()

        bufs = ((rows_a, sga), (rows_b, sgb), (rows_c, sgc))

        def group(gi, carry):
            pltpu.sync_copy(ei_hbm.at[0, w, gi], srcv)
            pltpu.sync_copy(ei_hbm.at[1, w, gi], dstv)
            g_start(0, rows_a, sga)
            g_start(1, rows_b, sgb)
            g_wait(rows_a, sga)
            s_start(0, rows_a)
            g_start(2, rows_c, sgc)

            def tri(k, c2):
                j0 = 3 * k + 1
                for d in range(3):
                    j = j0 + d
                    buf, sem = bufs[(1 + d) % 3]
                    nbuf, nsem = bufs[d % 3]
                    g_wait(buf, sem)
                    s_start(j, buf)
                    s_wait()
                    if d < 2:
                        g_start(j + 2, nbuf, nsem)
                    else:
                        @pl.when(j + 2 < CPG)
                        def _():
                            g_start(j + 2, nbuf, nsem)

                return c2

            lax.fori_loop(0, (CPG - 2) // 3, tri, 0)
            ebuf, esem = bufs[(CPG - 1) % 3]
            g_wait(ebuf, esem)
            s_start(CPG - 1, ebuf)
            s_wait()
            s_wait()
            return carry

        lax.fori_loop(0, NGRP, group, 0)
        plsc.subcore_barrier()
        pltpu.sync_copy(acc_sh.at[pl.ds(s * RPT, RPT)],
                        out_hbm.at[c].at[pl.ds(s * RPT, RPT)])

    return scat_kernel(h, ei5, z128)


def _mm0(x, W, cnt):
    def body(x_ref, w_ref, cnt_ref, hp_ref, dinv_ref):
        deg = cnt_ref[0, :, 0:1] + cnt_ref[1, :, 0:1] + 1.0
        dv = lax.rsqrt(deg)
        h = jnp.dot(x_ref[...], w_ref[...], preferred_element_type=F32)
        hp_ref[...] = h * dv
        dinv_ref[...] = jnp.broadcast_to(dv, (NB, 16))

    return pl.pallas_call(
        body,
        grid=(NBLK,),
        in_specs=[
            pl.BlockSpec((NB, D), lambda i: (i, 0)),
            pl.BlockSpec((D, D), lambda i: (0, 0)),
            pl.BlockSpec((NC, NB, 8), lambda i: (0, i, 0)),
        ],
        out_specs=[
            pl.BlockSpec((NB, D), lambda i: (i, 0)),
            pl.BlockSpec((NB, 16), lambda i: (i, 0)),
        ],
        out_shape=[
            jax.ShapeDtypeStruct((N, D), F32),
            jax.ShapeDtypeStruct((N, 16), F32),
        ],
    )(x, W, cnt)


def _combine_mm(acc, hp, dinv, b, g, bt, W):
    def body(acc_ref, hp_ref, dinv_ref, b_ref, g_ref, bt_ref, w_ref,
             out_ref, t_sc, st_sc):
        p = pl.program_id(0)
        i = pl.program_id(1)

        @pl.when((p == 0) & (i == 0))
        def _():
            st_sc[...] = jnp.zeros_like(st_sc)

        @pl.when(p == 0)
        def _():
            dv = dinv_ref[:, 0:1]
            o = dv * (acc_ref[0] + acc_ref[1] + hp_ref[...]) + b_ref[...]
            t = jnp.where(o > 0, o, jnp.exp(o) - 1.0)
            t_sc[i] = t
            st_sc[0:1, :] += jnp.sum(t, axis=0, keepdims=True)
            st_sc[1:2, :] += jnp.sum(t * t, axis=0, keepdims=True)

        @pl.when(p == 1)
        def _():
            m = st_sc[0:1, :] / N
            v = st_sc[1:2, :] / N - m * m
            sc = g_ref[...] * lax.rsqrt(v + EPS)
            sh = bt_ref[...] - m * sc
            y = t_sc[i] * sc + sh
            h = jnp.dot(y, w_ref[...], preferred_element_type=F32)
            out_ref[...] = h * dinv_ref[:, 0:1]

    return pl.pallas_call(
        body,
        grid=(2, NBLK),
        in_specs=[
            pl.BlockSpec((NC, NB, D), lambda p, i: (0, i * (1 - p), 0)),
            pl.BlockSpec((NB, D), lambda p, i: (i * (1 - p), 0)),
            pl.BlockSpec((NB, 16), lambda p, i: (i, 0)),
            pl.BlockSpec((1, D), lambda p, i: (0, 0)),
            pl.BlockSpec((1, D), lambda p, i: (0, 0)),
            pl.BlockSpec((1, D), lambda p, i: (0, 0)),
            pl.BlockSpec((D, D), lambda p, i: (0, 0)),
        ],
        out_specs=pl.BlockSpec((NB, D), lambda p, i: (i, 0)),
        out_shape=jax.ShapeDtypeStruct((N, D), F32),
        scratch_shapes=[
            pltpu.VMEM((NBLK, NB, D), F32),
            pltpu.VMEM((8, D), F32),
        ],
    )(acc, hp, dinv, b, g, bt, W)


def _combine_pool(acc, hp, dinv, b, g, bt, batch2, l1w, l1b, l2w, l2b):
    def body(acc_ref, hp_ref, dinv_ref, b_ref, g_ref, bt_ref, batch_ref,
             l1w_ref, l1b_ref, l2w_ref, l2b_ref, out_ref,
             t_sc, st_sc, pool_sc, cnt_sc):
        p = pl.program_id(0)
        i = pl.program_id(1)

        @pl.when((p == 0) & (i == 0))
        def _():
            st_sc[...] = jnp.zeros_like(st_sc)
            pool_sc[...] = jnp.zeros_like(pool_sc)
            cnt_sc[...] = jnp.zeros_like(cnt_sc)

        @pl.when(p == 0)
        def _():
            dv = dinv_ref[:, 0:1]
            o = dv * (acc_ref[0] + acc_ref[1] + hp_ref[...]) + b_ref[...]
            t = jnp.where(o > 0, o, jnp.exp(o) - 1.0)
            t_sc[i] = t
            st_sc[0:1, :] += jnp.sum(t, axis=0, keepdims=True)
            st_sc[1:2, :] += jnp.sum(t * t, axis=0, keepdims=True)

        @pl.when(p == 1)
        def _():
            m = st_sc[0:1, :] / N
            v = st_sc[1:2, :] / N - m * m
            sc = g_ref[...] * lax.rsqrt(v + EPS)
            sh = bt_ref[...] - m * sc
            y = t_sc[i] * sc + sh
            gids = lax.broadcasted_iota(jnp.int32, (G, NB), 0)
            bb = batch_ref[...].reshape(1, NB)
            oh = (bb == gids).astype(F32)
            pool_sc[...] += lax.dot_general(
                oh, y, (((1,), (0,)), ((), ())), preferred_element_type=F32)
            cnt_sc[...] += lax.dot_general(
                oh, jnp.ones((NB, 1), F32), (((1,), (0,)), ((), ())),
                preferred_element_type=F32)

        @pl.when((p == 1) & (i == NBLK - 1))
        def _():
            cnt = jnp.maximum(cnt_sc[...], 1.0)
            pooled = pool_sc[...] / cnt
            z = jnp.dot(pooled, l1w_ref[...], preferred_element_type=F32)
            z = jnp.maximum(z + l1b_ref[...], 0.0)
            z2 = jnp.dot(z, l2w_ref[...], preferred_element_type=F32)
            z2 = z2 + l2b_ref[...]
            mx = jnp.max(z2, axis=-1, keepdims=True)
            lse = jnp.log(jnp.sum(jnp.exp(z2 - mx), axis=-1, keepdims=True)) + mx
            out_ref[...] = z2 - lse

    return pl.pallas_call(
        body,
        grid=(2, NBLK),
        in_specs=[
            pl.BlockSpec((NC, NB, D), lambda p, i: (0, i * (1 - p), 0)),
            pl.BlockSpec((NB, D), lambda p, i: (i * (1 - p), 0)),
            pl.BlockSpec((NB, 16), lambda p, i: (i, 0)),
            pl.BlockSpec((1, D), lambda p, i: (0, 0)),
            pl.BlockSpec((1, D), lambda p, i: (0, 0)),
            pl.BlockSpec((1, D), lambda p, i: (0, 0)),
            pl.BlockSpec((1, 1, NB), lambda p, i: (i, 0, 0)),
            pl.BlockSpec((D, D), lambda p, i: (0, 0)),
            pl.BlockSpec((1, D), lambda p, i: (0, 0)),
            pl.BlockSpec((D, C), lambda p, i: (0, 0)),
            pl.BlockSpec((1, C), lambda p, i: (0, 0)),
        ],
        out_specs=pl.BlockSpec((G, C), lambda p, i: (0, 0)),
        out_shape=jax.ShapeDtypeStruct((G, C), F32),
        scratch_shapes=[
            pltpu.VMEM((NBLK, NB, D), F32),
            pltpu.VMEM((8, D), F32),
            pltpu.VMEM((G, D), F32),
            pltpu.VMEM((G, 1), F32),
        ],
    )(acc, hp, dinv, b, g, bt, batch2, l1w, l1b, l2w, l2b)


def kernel(x, edge_index, batch, W0, b0, g0, bt0, W1, b1, g1, bt1,
           W2, b2, g2, bt2, l1w, l1b, l2w, l2b):
    ei5 = edge_index.reshape(2, NW, NGRP, CPG, K)
    ones16 = jnp.concatenate(
        [jnp.ones((K, 1), F32), jnp.zeros((K, 7), F32)], axis=1)
    z16 = jnp.zeros((RPT, 8), F32)
    z128 = jnp.zeros((RPT, D), F32)
    cnt = _deg_sc(ei5, ones16, z16)
    hp, dinv = _mm0(x, W0, cnt)

    acc = _scatter_sc(hp, ei5, z128)
    hp = _combine_mm(acc, hp, dinv, b0.reshape(1, D), g0.reshape(1, D),
                     bt0.reshape(1, D), W1)
    acc = _scatter_sc(hp, ei5, z128)
    hp = _combine_mm(acc, hp, dinv, b1.reshape(1, D), g1.reshape(1, D),
                     bt1.reshape(1, D), W2)
    acc = _scatter_sc(hp, ei5, z128)
    return _combine_pool(acc, hp, dinv, b2.reshape(1, D), g2.reshape(1, D),
                         bt2.reshape(1, D), batch.reshape(NBLK, 1, NB), l1w, l1b.reshape(1, D),
                         l2w, l2b.reshape(1, C))
